# baseline (speedup 1.0000x reference)
import functools

import jax
import jax.numpy as jnp
from jax import lax
from jax.experimental import pallas as pl
from jax.experimental.pallas import tpu as pltpu

N = 24000
M = 64
COARSE = 512
NUM_MASK = 19
KNN = 16
SUB = 8
LANES = N // SUB
BLK = 480
NBLK = N // BLK
F_INF = 3e38
IBIG = 1 << 30

_HIGH = jax.lax.Precision.HIGHEST


def _dot(a, b):
    return jax.lax.dot_general(
        a, b, (((1,), (0,)), ((), ())),
        precision=_HIGH, preferred_element_type=jnp.float32)


def _dot_bf(a, b):
    return jax.lax.dot_general(
        a.astype(jnp.bfloat16), b.astype(jnp.bfloat16), (((1,), (0,)), ((), ())),
        preferred_element_type=jnp.float32)


def _fps_body(pt_ref, p0_ref, pc_ref, dist_ref):
    dist_ref[...] = jnp.full((SUB, LANES), 1e10, jnp.float32)
    ii = (lax.broadcasted_iota(jnp.int32, (SUB, LANES), 0) * LANES
          + lax.broadcasted_iota(jnp.int32, (SUB, LANES), 1))

    def body(k, last):
        row = p0_ref[pl.ds(last, 1), :]
        pc_ref[pl.ds(k, 1), :] = row
        cx, cy, cz = row[:, 0:1], row[:, 1:2], row[:, 2:3]
        dx = pt_ref[0] - cx
        dy = pt_ref[1] - cy
        dz = pt_ref[2] - cz
        d = (dx * dx + dy * dy) + dz * dz
        dist = jnp.minimum(dist_ref[...], d)
        dist_ref[...] = dist
        m = jnp.max(dist)
        nxt = jnp.min(jnp.where(dist == m, ii, IBIG))
        return nxt

    last = lax.fori_loop(0, COARSE - 1, body, jnp.int32(0), unroll=2)
    pc_ref[pl.ds(COARSE - 1, 1), :] = p0_ref[pl.ds(last, 1), :]


def _run_fps(pt3, p0):
    return pl.pallas_call(
        _fps_body,
        out_shape=jax.ShapeDtypeStruct((COARSE, 3), jnp.float32),
        scratch_shapes=[pltpu.VMEM((SUB, LANES), jnp.float32)],
    )(pt3, p0)


def _cluster_body(p_ref, x_ref, pcT_ref, fl_ref, xm_ref, mk_ref):
    p = p_ref[...]
    px, py, pz = p[:, 0:1], p[:, 1:2], p[:, 2:3]
    sp = (px * px + py * py) + pz * pz
    cx = pcT_ref[0:1, 0:M]
    cy = pcT_ref[1:2, 0:M]
    cz = pcT_ref[2:3, 0:M]
    sc = (cx * cx + cy * cy) + cz * cz
    dot = _dot_bf(p, pcT_ref[:, 0:M])
    d2 = (sp + sc) - 2.0 * dot
    rmin = jnp.min(d2, axis=1, keepdims=True)
    i64 = lax.broadcasted_iota(jnp.int32, d2.shape, 1)
    cl = jnp.min(jnp.where(d2 == rmin, i64, IBIG), axis=1, keepdims=True)
    onehot = i64 == cl
    maskf = jnp.max(jnp.where(onehot, fl_ref[0:1, :], 0.0), axis=1,
                    keepdims=True)
    mk_ref[...] = maskf
    xm_ref[...] = jnp.where(maskf > 0.0, 0.0, x_ref[...])


def _run_cluster(p0, x0, pcT, flags):
    blk = 3000
    grid = N // blk
    return pl.pallas_call(
        _cluster_body,
        grid=(grid,),
        in_specs=[
            pl.BlockSpec((blk, 3), lambda i: (i, 0)),
            pl.BlockSpec((blk, 6), lambda i: (i, 0)),
            pl.BlockSpec((3, COARSE), lambda i: (0, 0)),
            pl.BlockSpec((1, M), lambda i: (0, 0)),
        ],
        out_specs=[
            pl.BlockSpec((blk, 6), lambda i: (i, 0)),
            pl.BlockSpec((blk, 1), lambda i: (i, 0)),
        ],
        out_shape=[
            jax.ShapeDtypeStruct((N, 6), jnp.float32),
            jax.ShapeDtypeStruct((N, 1), jnp.float32),
        ],
    )(p0, x0, pcT, flags)


G = 64
GL = N // G
TOPG = 4


def _knn_body(pc_ref, pt_ref, ptb_ref, xm_ref, we_ref, be_ref, w1_ref, fc1_ref,
              d2_ref, x_ref, t_ref, it_ref, sp_ref):
    @pl.when(pl.program_id(0) == 0)
    def _():
        px = pt_ref[0:1, :]
        py = pt_ref[1:2, :]
        pz = pt_ref[2:3, :]
        sp_ref[...] = (px * px + py * py) + pz * pz

    c = pc_ref[...]
    cx, cy, cz = c[:, 0:1], c[:, 1:2], c[:, 2:3]
    sc = (cx * cx + cy * cy) + cz * cz
    dot = jax.lax.dot_general(
        c.astype(jnp.bfloat16), ptb_ref[...], (((1,), (0,)), ((), ())),
        preferred_element_type=jnp.float32)
    d2 = (sc + sp_ref[...]) - 2.0 * dot
    d2_ref[...] = d2

    x_ref[...] = d2.reshape(8, G, GL)
    cands = []
    for j in range(TOPG):
        x = x_ref[...]
        m = jnp.min(x, axis=2, keepdims=True)
        if j + 1 < TOPG:
            x_ref[...] = jnp.where(x == m, F_INF, x)
        cands.append(m)
    cand = jnp.concatenate(cands, axis=2).reshape(8, G * TOPG)

    t = cand
    for j in range(KNN):
        m = jnp.min(cand, axis=1, keepdims=True)
        if j + 1 < KNN:
            cand = jnp.where(cand == m, F_INF, cand)
        t = m
    t_ref[...] = t
    it_ref[...] = jnp.full((8, 1), IBIG - 1, jnp.int32)

    cnt = jnp.sum((d2 <= t).astype(jnp.float32), axis=1, keepdims=True)
    bad = jnp.max(jnp.abs(cnt - KNN)) > 0.0

    @pl.when(bad)
    def _():
        x_ref[...] = d2_ref[...].reshape(8, G, GL)
        ig = (lax.broadcasted_iota(jnp.int32, (8, G, GL), 1) * GL
              + lax.broadcasted_iota(jnp.int32, (8, G, GL), 2))

        def fb_body(_, carry):
            x = x_ref[...]
            rmin = jnp.min(x, axis=(1, 2), keepdims=True)
            imin = jnp.min(jnp.where(x == rmin, ig, IBIG), axis=(1, 2),
                           keepdims=True)
            x_ref[...] = jnp.where(ig == imin, F_INF, x)
            return rmin[:, 0, :], imin[:, 0, :]
        t_fb, it_fb = lax.fori_loop(
            0, KNN, fb_body,
            (jnp.zeros((8, 1), jnp.float32), jnp.zeros((8, 1), jnp.int32)))
        t_ref[...] = t_fb
        it_ref[...] = it_fb

    iN = lax.broadcasted_iota(jnp.int32, (8, N), 1)
    d2o = d2_ref[...]
    sel = (d2o < t_ref[...]) | ((d2o == t_ref[...]) & (iN <= it_ref[...]))
    A = jnp.where(sel, 1.0 / KNN, 0.0)
    agg = _dot(A, xm_ref[...])
    f = jnp.maximum(_dot(agg, we_ref[...]) + be_ref[0:1, :], 0.0)
    fc1_ref[...] = _dot(f, w1_ref[...])


def _run_knn(pc, pt2, ptb, xm, W_enc, b_enc, w1):
    return pl.pallas_call(
        _knn_body,
        grid=(COARSE // 8,),
        in_specs=[
            pl.BlockSpec((8, 3), lambda i: (i, 0)),
            pl.BlockSpec((3, N), lambda i: (0, 0)),
            pl.BlockSpec((3, N), lambda i: (0, 0)),
            pl.BlockSpec((N, 6), lambda i: (0, 0)),
            pl.BlockSpec((6, 512), lambda i: (0, 0)),
            pl.BlockSpec((1, 512), lambda i: (0, 0)),
            pl.BlockSpec((512, 256), lambda i: (0, 0)),
        ],
        out_specs=pl.BlockSpec((8, 256), lambda i: (i, 0)),
        out_shape=jax.ShapeDtypeStruct((COARSE, 256), jnp.float32),
        scratch_shapes=[pltpu.VMEM((8, N), jnp.float32),
                        pltpu.VMEM((8, G, GL), jnp.float32),
                        pltpu.VMEM((8, 1), jnp.float32),
                        pltpu.VMEM((8, 1), jnp.int32),
                        pltpu.VMEM((1, N), jnp.float32)],
    )(pc, pt2, ptb, xm, W_enc, b_enc, w1)


def _interp_body(p_ref, pcT_ref, fc1_ref, b1_ref, z1_ref, s1_ref, q1_ref):
    @pl.when(pl.program_id(0) == 0)
    def _():
        s1_ref[...] = jnp.zeros_like(s1_ref)
        q1_ref[...] = jnp.zeros_like(q1_ref)

    p = p_ref[...]
    px, py, pz = p[:, 0:1], p[:, 1:2], p[:, 2:3]
    sp = (px * px + py * py) + pz * pz
    cx = pcT_ref[0:1, :]
    cy = pcT_ref[1:2, :]
    cz = pcT_ref[2:3, :]
    sc = (cx * cx + cy * cy) + cz * cz
    dot = _dot_bf(p, pcT_ref[...])
    d2 = (sp + sc) - 2.0 * dot

    d0 = d2
    ds_ = []
    for j in range(3):
        rmin = jnp.min(d2, axis=1, keepdims=True)
        ds_.append(rmin)
        if j < 2:
            d2 = jnp.where(d2 == rmin, F_INF, d2)

    w = [1.0 / (jnp.sqrt(jnp.maximum(d, 1e-12)) + 1e-8) for d in ds_]
    wsum = (w[0] + w[1]) + w[2]
    wmat = (jnp.where(d0 == ds_[0], w[0] / wsum, 0.0)
            + jnp.where(d0 == ds_[1], w[1] / wsum, 0.0)
            + jnp.where(d0 == ds_[2], w[2] / wsum, 0.0))
    z1 = _dot(wmat, fc1_ref[...]) + b1_ref[0:1, :]
    z1_ref[...] = z1
    s1_ref[...] += jnp.sum(z1, axis=0, keepdims=True)
    q1_ref[...] += jnp.sum(z1 * z1, axis=0, keepdims=True)


def _run_interp(p0, pcT, fc1, b1):
    return pl.pallas_call(
        _interp_body,
        grid=(NBLK,),
        in_specs=[
            pl.BlockSpec((BLK, 3), lambda i: (i, 0)),
            pl.BlockSpec((3, COARSE), lambda i: (0, 0)),
            pl.BlockSpec((COARSE, 256), lambda i: (0, 0)),
            pl.BlockSpec((1, 256), lambda i: (0, 0)),
        ],
        out_specs=[
            pl.BlockSpec((BLK, 256), lambda i: (i, 0)),
            pl.BlockSpec((1, 256), lambda i: (0, 0)),
            pl.BlockSpec((1, 256), lambda i: (0, 0)),
        ],
        out_shape=[
            jax.ShapeDtypeStruct((N, 256), jnp.float32),
            jax.ShapeDtypeStruct((1, 256), jnp.float32),
            jax.ShapeDtypeStruct((1, 256), jnp.float32),
        ],
    )(p0, pcT, fc1, b1)


def _bn_mm_body(z_ref, s_ref, q_ref, g_ref, be_ref, w_ref, bb_ref,
                o_ref, so_ref, qo_ref):
    @pl.when(pl.program_id(0) == 0)
    def _():
        so_ref[...] = jnp.zeros_like(so_ref)
        qo_ref[...] = jnp.zeros_like(qo_ref)

    inv_n = jnp.float32(1.0 / N)
    mu = s_ref[...] * inv_n
    var = q_ref[...] * inv_n - mu * mu
    z = z_ref[...]
    h = (z - mu) / jnp.sqrt(var + 1e-5) * g_ref[0:1, :] + be_ref[0:1, :]
    h = jnp.maximum(h, 0.0)
    o = _dot(h, w_ref[...]) + bb_ref[0:1, :]
    o_ref[...] = o
    so_ref[...] += jnp.sum(o, axis=0, keepdims=True)
    qo_ref[...] += jnp.sum(o * o, axis=0, keepdims=True)


def _run_bn_mm(z, s, q, g, beta, w, b, din, dout):
    return pl.pallas_call(
        _bn_mm_body,
        grid=(NBLK,),
        in_specs=[
            pl.BlockSpec((BLK, din), lambda i: (i, 0)),
            pl.BlockSpec((1, din), lambda i: (0, 0)),
            pl.BlockSpec((1, din), lambda i: (0, 0)),
            pl.BlockSpec((1, din), lambda i: (0, 0)),
            pl.BlockSpec((1, din), lambda i: (0, 0)),
            pl.BlockSpec((din, dout), lambda i: (0, 0)),
            pl.BlockSpec((1, dout), lambda i: (0, 0)),
        ],
        out_specs=[
            pl.BlockSpec((BLK, dout), lambda i: (i, 0)),
            pl.BlockSpec((1, dout), lambda i: (0, 0)),
            pl.BlockSpec((1, dout), lambda i: (0, 0)),
        ],
        out_shape=[
            jax.ShapeDtypeStruct((N, dout), jnp.float32),
            jax.ShapeDtypeStruct((1, dout), jnp.float32),
            jax.ShapeDtypeStruct((1, dout), jnp.float32),
        ],
    )(z, s, q, g, beta, w, b)


def _final_body(z_ref, s_ref, q_ref, g_ref, be_ref, w_ref, bb_ref,
                xm_ref, mk_ref, xr_ref, ls_ref, ms_ref):
    @pl.when(pl.program_id(0) == 0)
    def _():
        ls_ref[...] = jnp.zeros_like(ls_ref)
        ms_ref[...] = jnp.zeros_like(ms_ref)

    inv_n = jnp.float32(1.0 / N)
    mu = s_ref[...] * inv_n
    var = q_ref[...] * inv_n - mu * mu
    h = (z_ref[...] - mu) / jnp.sqrt(var + 1e-5) * g_ref[0:1, :] + be_ref[0:1, :]
    h = jnp.maximum(h, 0.0)
    xr = _dot(h, w_ref[...]) + bb_ref[0:1, :]
    xm = xm_ref[...]
    mk = mk_ref[...]
    l1 = jnp.abs(xr - xm) * mk
    ls_ref[...] += jnp.sum(l1, axis=(0, 1), keepdims=True)
    ms_ref[...] += jnp.sum(mk, axis=(0, 1), keepdims=True)
    xr_ref[...] = jnp.where(mk > 0.0, xr, xm)


def _run_final(z2, s2, q2, g2, beta2, w3, b3, xm, mk):
    return pl.pallas_call(
        _final_body,
        grid=(NBLK,),
        in_specs=[
            pl.BlockSpec((BLK, 128), lambda i: (i, 0)),
            pl.BlockSpec((1, 128), lambda i: (0, 0)),
            pl.BlockSpec((1, 128), lambda i: (0, 0)),
            pl.BlockSpec((1, 128), lambda i: (0, 0)),
            pl.BlockSpec((1, 128), lambda i: (0, 0)),
            pl.BlockSpec((128, 6), lambda i: (0, 0)),
            pl.BlockSpec((1, 6), lambda i: (0, 0)),
            pl.BlockSpec((BLK, 6), lambda i: (i, 0)),
            pl.BlockSpec((BLK, 1), lambda i: (i, 0)),
        ],
        out_specs=[
            pl.BlockSpec((BLK, 6), lambda i: (i, 0)),
            pl.BlockSpec((1, 1), lambda i: (0, 0)),
            pl.BlockSpec((1, 1), lambda i: (0, 0)),
        ],
        out_shape=[
            jax.ShapeDtypeStruct((N, 6), jnp.float32),
            jax.ShapeDtypeStruct((1, 1), jnp.float32),
            jax.ShapeDtypeStruct((1, 1), jnp.float32),
        ],
    )(z2, s2, q2, g2, beta2, w3, b3, xm, mk)


def kernel(point, features, W_enc, b_enc, w1, b1, g1, beta1, w2, b2, g2,
           beta2, w3, b3):
    p0 = point.reshape(-1, 3)
    x0 = features.reshape(-1, 6)
    pt2 = p0.T
    pt3 = pt2.reshape(3, SUB, LANES)

    masked_clusters = jax.random.permutation(
        jax.random.key(1), M)[:NUM_MASK].astype(jnp.int32)
    flags = jnp.isin(jnp.arange(M, dtype=jnp.int32),
                     masked_clusters).astype(jnp.float32).reshape(1, M)

    pc = _run_fps(pt3, p0)
    pcT = pc.T

    xm, mk = _run_cluster(p0, x0, pcT, flags)
    fc1 = _run_knn(pc, pt2, pt2.astype(jnp.bfloat16), xm, W_enc,
                   b_enc.reshape(1, -1), w1)
    z1, s1, q1 = _run_interp(p0, pcT, fc1, b1.reshape(1, -1))
    z2, s2, q2 = _run_bn_mm(z1, s1, q1, g1.reshape(1, -1), beta1.reshape(1, -1),
                            w2, b2.reshape(1, -1), 256, 128)
    xrec, lsum, msum = _run_final(z2, s2, q2, g2.reshape(1, -1),
                                  beta2.reshape(1, -1), w3, b3.reshape(1, -1),
                                  xm, mk)

    loss = lsum[0, 0] / (msum[0, 0] * 6.0)
    mask = mk[:, 0] > 0.5
    visible = jnp.logical_not(mask)
    return (loss, xrec, mask, visible, xm)


# speedup vs baseline: 1.4341x; 1.4341x over previous
import functools

import jax
import jax.numpy as jnp
from jax import lax
from jax.experimental import pallas as pl
from jax.experimental.pallas import tpu as pltpu

N = 24000
M = 64
COARSE = 512
NUM_MASK = 19
KNN = 16
SUB = 8
LANES = N // SUB
BLK = 480
NBLK = N // BLK
F_INF = 3e38
IBIG = 1 << 30

_HIGH = jax.lax.Precision.HIGHEST


def _dot(a, b):
    return jax.lax.dot_general(
        a, b, (((1,), (0,)), ((), ())),
        precision=_HIGH, preferred_element_type=jnp.float32)


def _dot_b(a, b):
    return jax.lax.dot_general(
        a, b, (((1,), (0,)), ((), ())), preferred_element_type=jnp.float32)


def _dot_bf(a, b):
    return jax.lax.dot_general(
        a.astype(jnp.bfloat16), b.astype(jnp.bfloat16), (((1,), (0,)), ((), ())),
        preferred_element_type=jnp.float32)


def _fps_body(pt_ref, p0_ref, pc_ref, dist_ref):
    dist_ref[...] = jnp.full((SUB, LANES), 1e10, jnp.float32)
    ii = (lax.broadcasted_iota(jnp.int32, (SUB, LANES), 0) * LANES
          + lax.broadcasted_iota(jnp.int32, (SUB, LANES), 1))

    def body(k, last):
        row = p0_ref[pl.ds(last, 1), :]
        pc_ref[pl.ds(k, 1), :] = row
        cx, cy, cz = row[:, 0:1], row[:, 1:2], row[:, 2:3]
        dx = pt_ref[0] - cx
        dy = pt_ref[1] - cy
        dz = pt_ref[2] - cz
        d = (dx * dx + dy * dy) + dz * dz
        dist = jnp.minimum(dist_ref[...], d)
        dist_ref[...] = dist
        m = jnp.max(dist)
        nxt = jnp.min(jnp.where(dist == m, ii, IBIG))
        return nxt

    last = lax.fori_loop(0, COARSE - 1, body, jnp.int32(0), unroll=2)
    pc_ref[pl.ds(COARSE - 1, 1), :] = p0_ref[pl.ds(last, 1), :]


def _run_fps(pt3, p0):
    return pl.pallas_call(
        _fps_body,
        out_shape=jax.ShapeDtypeStruct((COARSE, 3), jnp.float32),
        scratch_shapes=[pltpu.VMEM((SUB, LANES), jnp.float32)],
    )(pt3, p0)


def _cluster_body(p_ref, x_ref, pcT_ref, fl_ref, xm_ref, mk_ref):
    p = p_ref[...]
    px, py, pz = p[:, 0:1], p[:, 1:2], p[:, 2:3]
    sp = (px * px + py * py) + pz * pz
    cx = pcT_ref[0:1, 0:M]
    cy = pcT_ref[1:2, 0:M]
    cz = pcT_ref[2:3, 0:M]
    sc = (cx * cx + cy * cy) + cz * cz
    dot = _dot_bf(p, pcT_ref[:, 0:M])
    d2 = (sp + sc) - 2.0 * dot
    rmin = jnp.min(d2, axis=1, keepdims=True)
    i64 = lax.broadcasted_iota(jnp.int32, d2.shape, 1)
    cl = jnp.min(jnp.where(d2 == rmin, i64, IBIG), axis=1, keepdims=True)
    onehot = i64 == cl
    maskf = jnp.max(jnp.where(onehot, fl_ref[0:1, :], 0.0), axis=1,
                    keepdims=True)
    mk_ref[...] = maskf
    xm_ref[...] = jnp.where(maskf > 0.0, 0.0, x_ref[...])


def _run_cluster(p0, x0, pcT, flags):
    blk = 3000
    grid = N // blk
    return pl.pallas_call(
        _cluster_body,
        grid=(grid,),
        in_specs=[
            pl.BlockSpec((blk, 3), lambda i: (i, 0)),
            pl.BlockSpec((blk, 6), lambda i: (i, 0)),
            pl.BlockSpec((3, COARSE), lambda i: (0, 0)),
            pl.BlockSpec((1, M), lambda i: (0, 0)),
        ],
        out_specs=[
            pl.BlockSpec((blk, 6), lambda i: (i, 0)),
            pl.BlockSpec((blk, 1), lambda i: (i, 0)),
        ],
        out_shape=[
            jax.ShapeDtypeStruct((N, 6), jnp.float32),
            jax.ShapeDtypeStruct((N, 1), jnp.float32),
        ],
    )(p0, x0, pcT, flags)


G = 64
GL = N // G
TOPG = 4


def _knn_body(pc_ref, pt_ref, ptb_ref, xm_ref, we_ref, be_ref, w1_ref, fc1_ref,
              d2_ref, x_ref, t_ref, it_ref, sp_ref):
    @pl.when(pl.program_id(0) == 0)
    def _():
        px = pt_ref[0:1, :]
        py = pt_ref[1:2, :]
        pz = pt_ref[2:3, :]
        sp_ref[...] = (px * px + py * py) + pz * pz

    c = pc_ref[...]
    cx, cy, cz = c[:, 0:1], c[:, 1:2], c[:, 2:3]
    sc = (cx * cx + cy * cy) + cz * cz
    dot = jax.lax.dot_general(
        c.astype(jnp.bfloat16), ptb_ref[...], (((1,), (0,)), ((), ())),
        preferred_element_type=jnp.float32)
    d2 = (sc + sp_ref[...]) - 2.0 * dot
    d2_ref[...] = d2

    x_ref[...] = d2.reshape(8, G, GL)
    cands = []
    for j in range(TOPG):
        x = x_ref[...]
        m = jnp.min(x, axis=2, keepdims=True)
        if j + 1 < TOPG:
            x_ref[...] = jnp.where(x == m, F_INF, x)
        cands.append(m)
    cand = jnp.concatenate(cands, axis=2).reshape(8, G * TOPG)

    t = cand
    for j in range(KNN):
        m = jnp.min(cand, axis=1, keepdims=True)
        if j + 1 < KNN:
            cand = jnp.where(cand == m, F_INF, cand)
        t = m
    t_ref[...] = t
    it_ref[...] = jnp.full((8, 1), IBIG - 1, jnp.int32)

    cnt = jnp.sum((d2 <= t).astype(jnp.float32), axis=1, keepdims=True)
    bad = jnp.max(jnp.abs(cnt - KNN)) > 0.0

    @pl.when(bad)
    def _():
        x_ref[...] = d2_ref[...].reshape(8, G, GL)
        ig = (lax.broadcasted_iota(jnp.int32, (8, G, GL), 1) * GL
              + lax.broadcasted_iota(jnp.int32, (8, G, GL), 2))

        def fb_body(_, carry):
            x = x_ref[...]
            rmin = jnp.min(x, axis=(1, 2), keepdims=True)
            imin = jnp.min(jnp.where(x == rmin, ig, IBIG), axis=(1, 2),
                           keepdims=True)
            x_ref[...] = jnp.where(ig == imin, F_INF, x)
            return rmin[:, 0, :], imin[:, 0, :]
        t_fb, it_fb = lax.fori_loop(
            0, KNN, fb_body,
            (jnp.zeros((8, 1), jnp.float32), jnp.zeros((8, 1), jnp.int32)))
        t_ref[...] = t_fb
        it_ref[...] = it_fb

    iN = lax.broadcasted_iota(jnp.int32, (8, N), 1)
    d2o = d2_ref[...]
    sel = (d2o < t_ref[...]) | ((d2o == t_ref[...]) & (iN <= it_ref[...]))
    A = jnp.where(sel, 1.0 / KNN, 0.0).astype(jnp.bfloat16)
    agg = _dot_b(A, xm_ref[...])
    f = jnp.maximum(_dot_b(agg.astype(jnp.bfloat16), we_ref[...])
                    + be_ref[0:1, :], 0.0)
    fc1_ref[...] = _dot_b(f.astype(jnp.bfloat16), w1_ref[...])


def _run_knn(pc, pt2, ptb, xm, W_enc, b_enc, w1):
    return pl.pallas_call(
        _knn_body,
        grid=(COARSE // 8,),
        in_specs=[
            pl.BlockSpec((8, 3), lambda i: (i, 0)),
            pl.BlockSpec((3, N), lambda i: (0, 0)),
            pl.BlockSpec((3, N), lambda i: (0, 0)),
            pl.BlockSpec((N, 6), lambda i: (0, 0)),
            pl.BlockSpec((6, 512), lambda i: (0, 0)),
            pl.BlockSpec((1, 512), lambda i: (0, 0)),
            pl.BlockSpec((512, 256), lambda i: (0, 0)),
        ],
        out_specs=pl.BlockSpec((8, 256), lambda i: (i, 0)),
        out_shape=jax.ShapeDtypeStruct((COARSE, 256), jnp.float32),
        scratch_shapes=[pltpu.VMEM((8, N), jnp.float32),
                        pltpu.VMEM((8, G, GL), jnp.float32),
                        pltpu.VMEM((8, 1), jnp.float32),
                        pltpu.VMEM((8, 1), jnp.int32),
                        pltpu.VMEM((1, N), jnp.float32)],
    )(pc, pt2, ptb, xm, W_enc, b_enc, w1)


def _interp_body(p_ref, pcT_ref, fc1_ref, b1_ref, z1_ref, s1_ref, q1_ref):
    @pl.when(pl.program_id(0) == 0)
    def _():
        s1_ref[...] = jnp.zeros_like(s1_ref)
        q1_ref[...] = jnp.zeros_like(q1_ref)

    p = p_ref[...]
    px, py, pz = p[:, 0:1], p[:, 1:2], p[:, 2:3]
    sp = (px * px + py * py) + pz * pz
    cx = pcT_ref[0:1, :]
    cy = pcT_ref[1:2, :]
    cz = pcT_ref[2:3, :]
    sc = (cx * cx + cy * cy) + cz * cz
    dot = _dot_bf(p, pcT_ref[...])
    d2 = (sp + sc) - 2.0 * dot

    d0 = d2
    ds_ = []
    for j in range(3):
        rmin = jnp.min(d2, axis=1, keepdims=True)
        ds_.append(rmin)
        if j < 2:
            d2 = jnp.where(d2 == rmin, F_INF, d2)

    w = [1.0 / (jnp.sqrt(jnp.maximum(d, 1e-12)) + 1e-8) for d in ds_]
    wsum = (w[0] + w[1]) + w[2]
    wmat = (jnp.where(d0 == ds_[0], w[0] / wsum, 0.0)
            + jnp.where(d0 == ds_[1], w[1] / wsum, 0.0)
            + jnp.where(d0 == ds_[2], w[2] / wsum, 0.0))
    z1 = _dot_b(wmat.astype(jnp.bfloat16), fc1_ref[...]) + b1_ref[0:1, :]
    z1_ref[...] = z1
    s1_ref[...] += jnp.sum(z1, axis=0, keepdims=True)
    q1_ref[...] += jnp.sum(z1 * z1, axis=0, keepdims=True)


def _run_interp(p0, pcT, fc1, b1):
    return pl.pallas_call(
        _interp_body,
        grid=(NBLK,),
        in_specs=[
            pl.BlockSpec((BLK, 3), lambda i: (i, 0)),
            pl.BlockSpec((3, COARSE), lambda i: (0, 0)),
            pl.BlockSpec((COARSE, 256), lambda i: (0, 0)),
            pl.BlockSpec((1, 256), lambda i: (0, 0)),
        ],
        out_specs=[
            pl.BlockSpec((BLK, 256), lambda i: (i, 0)),
            pl.BlockSpec((1, 256), lambda i: (0, 0)),
            pl.BlockSpec((1, 256), lambda i: (0, 0)),
        ],
        out_shape=[
            jax.ShapeDtypeStruct((N, 256), jnp.float32),
            jax.ShapeDtypeStruct((1, 256), jnp.float32),
            jax.ShapeDtypeStruct((1, 256), jnp.float32),
        ],
    )(p0, pcT, fc1, b1)


def _bn_mm_body(z_ref, s_ref, q_ref, g_ref, be_ref, w_ref, bb_ref,
                o_ref, so_ref, qo_ref):
    @pl.when(pl.program_id(0) == 0)
    def _():
        so_ref[...] = jnp.zeros_like(so_ref)
        qo_ref[...] = jnp.zeros_like(qo_ref)

    inv_n = jnp.float32(1.0 / N)
    mu = s_ref[...] * inv_n
    var = q_ref[...] * inv_n - mu * mu
    z = z_ref[...]
    h = (z - mu) / jnp.sqrt(var + 1e-5) * g_ref[0:1, :] + be_ref[0:1, :]
    h = jnp.maximum(h, 0.0)
    o = _dot_b(h.astype(jnp.bfloat16), w_ref[...]) + bb_ref[0:1, :]
    o_ref[...] = o
    so_ref[...] += jnp.sum(o, axis=0, keepdims=True)
    qo_ref[...] += jnp.sum(o * o, axis=0, keepdims=True)


def _run_bn_mm(z, s, q, g, beta, w, b, din, dout):
    return pl.pallas_call(
        _bn_mm_body,
        grid=(NBLK,),
        in_specs=[
            pl.BlockSpec((BLK, din), lambda i: (i, 0)),
            pl.BlockSpec((1, din), lambda i: (0, 0)),
            pl.BlockSpec((1, din), lambda i: (0, 0)),
            pl.BlockSpec((1, din), lambda i: (0, 0)),
            pl.BlockSpec((1, din), lambda i: (0, 0)),
            pl.BlockSpec((din, dout), lambda i: (0, 0)),
            pl.BlockSpec((1, dout), lambda i: (0, 0)),
        ],
        out_specs=[
            pl.BlockSpec((BLK, dout), lambda i: (i, 0)),
            pl.BlockSpec((1, dout), lambda i: (0, 0)),
            pl.BlockSpec((1, dout), lambda i: (0, 0)),
        ],
        out_shape=[
            jax.ShapeDtypeStruct((N, dout), jnp.float32),
            jax.ShapeDtypeStruct((1, dout), jnp.float32),
            jax.ShapeDtypeStruct((1, dout), jnp.float32),
        ],
    )(z, s, q, g, beta, w, b)


def _final_body(z_ref, s_ref, q_ref, g_ref, be_ref, w_ref, bb_ref,
                xm_ref, mk_ref, xr_ref, ls_ref, ms_ref):
    @pl.when(pl.program_id(0) == 0)
    def _():
        ls_ref[...] = jnp.zeros_like(ls_ref)
        ms_ref[...] = jnp.zeros_like(ms_ref)

    inv_n = jnp.float32(1.0 / N)
    mu = s_ref[...] * inv_n
    var = q_ref[...] * inv_n - mu * mu
    h = (z_ref[...] - mu) / jnp.sqrt(var + 1e-5) * g_ref[0:1, :] + be_ref[0:1, :]
    h = jnp.maximum(h, 0.0)
    xr = _dot_b(h.astype(jnp.bfloat16), w_ref[...]) + bb_ref[0:1, :]
    xm = xm_ref[...]
    mk = mk_ref[...]
    l1 = jnp.abs(xr - xm) * mk
    ls_ref[...] += jnp.sum(l1, axis=(0, 1), keepdims=True)
    ms_ref[...] += jnp.sum(mk, axis=(0, 1), keepdims=True)
    xr_ref[...] = jnp.where(mk > 0.0, xr, xm)


def _run_final(z2, s2, q2, g2, beta2, w3, b3, xm, mk):
    return pl.pallas_call(
        _final_body,
        grid=(NBLK,),
        in_specs=[
            pl.BlockSpec((BLK, 128), lambda i: (i, 0)),
            pl.BlockSpec((1, 128), lambda i: (0, 0)),
            pl.BlockSpec((1, 128), lambda i: (0, 0)),
            pl.BlockSpec((1, 128), lambda i: (0, 0)),
            pl.BlockSpec((1, 128), lambda i: (0, 0)),
            pl.BlockSpec((128, 6), lambda i: (0, 0)),
            pl.BlockSpec((1, 6), lambda i: (0, 0)),
            pl.BlockSpec((BLK, 6), lambda i: (i, 0)),
            pl.BlockSpec((BLK, 1), lambda i: (i, 0)),
        ],
        out_specs=[
            pl.BlockSpec((BLK, 6), lambda i: (i, 0)),
            pl.BlockSpec((1, 1), lambda i: (0, 0)),
            pl.BlockSpec((1, 1), lambda i: (0, 0)),
        ],
        out_shape=[
            jax.ShapeDtypeStruct((N, 6), jnp.float32),
            jax.ShapeDtypeStruct((1, 1), jnp.float32),
            jax.ShapeDtypeStruct((1, 1), jnp.float32),
        ],
    )(z2, s2, q2, g2, beta2, w3, b3, xm, mk)


def kernel(point, features, W_enc, b_enc, w1, b1, g1, beta1, w2, b2, g2,
           beta2, w3, b3):
    p0 = point.reshape(-1, 3)
    x0 = features.reshape(-1, 6)
    pt2 = p0.T
    pt3 = pt2.reshape(3, SUB, LANES)

    masked_clusters = jax.random.permutation(
        jax.random.key(1), M)[:NUM_MASK].astype(jnp.int32)
    flags = jnp.isin(jnp.arange(M, dtype=jnp.int32),
                     masked_clusters).astype(jnp.float32).reshape(1, M)

    pc = _run_fps(pt3, p0)
    pcT = pc.T

    bf = jnp.bfloat16
    xm, mk = _run_cluster(p0, x0, pcT, flags)
    fc1 = _run_knn(pc, pt2, pt2.astype(bf), xm.astype(bf), W_enc.astype(bf),
                   b_enc.reshape(1, -1), w1.astype(bf))
    z1, s1, q1 = _run_interp(p0, pcT, fc1.astype(bf), b1.reshape(1, -1))
    z2, s2, q2 = _run_bn_mm(z1, s1, q1, g1.reshape(1, -1), beta1.reshape(1, -1),
                            w2.astype(bf), b2.reshape(1, -1), 256, 128)
    xrec, lsum, msum = _run_final(z2, s2, q2, g2.reshape(1, -1),
                                  beta2.reshape(1, -1), w3.astype(bf),
                                  b3.reshape(1, -1), xm, mk)

    loss = lsum[0, 0] / (msum[0, 0] * 6.0)
    mask = mk[:, 0] > 0.5
    visible = jnp.logical_not(mask)
    return (loss, xrec, mask, visible, xm)


# speedup vs baseline: 1.4352x; 1.0008x over previous
import functools

import jax
import jax.numpy as jnp
from jax import lax
from jax.experimental import pallas as pl
from jax.experimental.pallas import tpu as pltpu

N = 24000
M = 64
COARSE = 512
NUM_MASK = 19
KNN = 16
SUB = 8
LANES = N // SUB
BLK = 480
NBLK = N // BLK
F_INF = 3e38
IBIG = 1 << 30

_HIGH = jax.lax.Precision.HIGHEST


def _dot(a, b):
    return jax.lax.dot_general(
        a, b, (((1,), (0,)), ((), ())),
        precision=_HIGH, preferred_element_type=jnp.float32)


def _dot_b(a, b):
    return jax.lax.dot_general(
        a, b, (((1,), (0,)), ((), ())), preferred_element_type=jnp.float32)


def _dot_bf(a, b):
    return jax.lax.dot_general(
        a.astype(jnp.bfloat16), b.astype(jnp.bfloat16), (((1,), (0,)), ((), ())),
        preferred_element_type=jnp.float32)


def _fps_body(pt_ref, p0_ref, pc_ref, dist_ref):
    dist_ref[...] = jnp.full((SUB, LANES), 1e10, jnp.float32)
    ii = (lax.broadcasted_iota(jnp.int32, (SUB, LANES), 0) * LANES
          + lax.broadcasted_iota(jnp.int32, (SUB, LANES), 1))

    def body(k, last):
        row = p0_ref[pl.ds(last, 1), :]
        pc_ref[pl.ds(k, 1), :] = row
        cx, cy, cz = row[:, 0:1], row[:, 1:2], row[:, 2:3]
        dx = pt_ref[0] - cx
        dy = pt_ref[1] - cy
        dz = pt_ref[2] - cz
        d = (dx * dx + dy * dy) + dz * dz
        dist = jnp.minimum(dist_ref[...], d)
        dist_ref[...] = dist
        m = jnp.max(dist, axis=(0, 1), keepdims=True)
        nxt = jnp.min(jnp.where(dist == m, ii, IBIG))
        return nxt

    last = lax.fori_loop(0, COARSE - 1, body, jnp.int32(0), unroll=2)
    pc_ref[pl.ds(COARSE - 1, 1), :] = p0_ref[pl.ds(last, 1), :]


def _run_fps(pt3, p0):
    return pl.pallas_call(
        _fps_body,
        out_shape=jax.ShapeDtypeStruct((COARSE, 3), jnp.float32),
        scratch_shapes=[pltpu.VMEM((SUB, LANES), jnp.float32)],
    )(pt3, p0)


def _cluster_body(p_ref, x_ref, pcT_ref, fl_ref, xm_ref, mk_ref):
    p = p_ref[...]
    px, py, pz = p[:, 0:1], p[:, 1:2], p[:, 2:3]
    sp = (px * px + py * py) + pz * pz
    cx = pcT_ref[0:1, 0:M]
    cy = pcT_ref[1:2, 0:M]
    cz = pcT_ref[2:3, 0:M]
    sc = (cx * cx + cy * cy) + cz * cz
    dot = _dot_bf(p, pcT_ref[:, 0:M])
    d2 = (sp + sc) - 2.0 * dot
    rmin = jnp.min(d2, axis=1, keepdims=True)
    i64 = lax.broadcasted_iota(jnp.int32, d2.shape, 1)
    cl = jnp.min(jnp.where(d2 == rmin, i64, IBIG), axis=1, keepdims=True)
    onehot = i64 == cl
    maskf = jnp.max(jnp.where(onehot, fl_ref[0:1, :], 0.0), axis=1,
                    keepdims=True)
    mk_ref[...] = maskf
    xm_ref[...] = jnp.where(maskf > 0.0, 0.0, x_ref[...])


def _run_cluster(p0, x0, pcT, flags):
    blk = 3000
    grid = N // blk
    return pl.pallas_call(
        _cluster_body,
        grid=(grid,),
        in_specs=[
            pl.BlockSpec((blk, 3), lambda i: (i, 0)),
            pl.BlockSpec((blk, 6), lambda i: (i, 0)),
            pl.BlockSpec((3, COARSE), lambda i: (0, 0)),
            pl.BlockSpec((1, M), lambda i: (0, 0)),
        ],
        out_specs=[
            pl.BlockSpec((blk, 6), lambda i: (i, 0)),
            pl.BlockSpec((blk, 1), lambda i: (i, 0)),
        ],
        out_shape=[
            jax.ShapeDtypeStruct((N, 6), jnp.float32),
            jax.ShapeDtypeStruct((N, 1), jnp.float32),
        ],
    )(p0, x0, pcT, flags)


G = 64
GL = N // G
TOPG = 4


def _knn_body(pc_ref, pt_ref, ptb_ref, xm_ref, we_ref, be_ref, w1_ref, fc1_ref,
              d2_ref, x_ref, t_ref, it_ref, sp_ref):
    @pl.when(pl.program_id(0) == 0)
    def _():
        px = pt_ref[0:1, :]
        py = pt_ref[1:2, :]
        pz = pt_ref[2:3, :]
        sp_ref[...] = (px * px + py * py) + pz * pz

    c = pc_ref[...]
    cx, cy, cz = c[:, 0:1], c[:, 1:2], c[:, 2:3]
    sc = (cx * cx + cy * cy) + cz * cz
    dot = jax.lax.dot_general(
        c.astype(jnp.bfloat16), ptb_ref[...], (((1,), (0,)), ((), ())),
        preferred_element_type=jnp.float32)
    d2 = (sc + sp_ref[...]) - 2.0 * dot
    d2_ref[...] = d2

    x_ref[...] = d2.reshape(8, G, GL)
    cands = []
    for j in range(TOPG):
        x = x_ref[...]
        m = jnp.min(x, axis=2, keepdims=True)
        if j + 1 < TOPG:
            x_ref[...] = jnp.where(x == m, F_INF, x)
        cands.append(m)
    cand = jnp.concatenate(cands, axis=2).reshape(8, G * TOPG)

    t = cand
    for j in range(KNN):
        m = jnp.min(cand, axis=1, keepdims=True)
        if j + 1 < KNN:
            cand = jnp.where(cand == m, F_INF, cand)
        t = m
    t_ref[...] = t
    it_ref[...] = jnp.full((8, 1), IBIG - 1, jnp.int32)

    cnt = jnp.sum((d2 <= t).astype(jnp.float32), axis=1, keepdims=True)
    bad = jnp.max(jnp.abs(cnt - KNN)) > 0.0

    @pl.when(bad)
    def _():
        x_ref[...] = d2_ref[...].reshape(8, G, GL)
        ig = (lax.broadcasted_iota(jnp.int32, (8, G, GL), 1) * GL
              + lax.broadcasted_iota(jnp.int32, (8, G, GL), 2))

        def fb_body(_, carry):
            x = x_ref[...]
            rmin = jnp.min(x, axis=(1, 2), keepdims=True)
            imin = jnp.min(jnp.where(x == rmin, ig, IBIG), axis=(1, 2),
                           keepdims=True)
            x_ref[...] = jnp.where(ig == imin, F_INF, x)
            return rmin[:, 0, :], imin[:, 0, :]
        t_fb, it_fb = lax.fori_loop(
            0, KNN, fb_body,
            (jnp.zeros((8, 1), jnp.float32), jnp.zeros((8, 1), jnp.int32)))
        t_ref[...] = t_fb
        it_ref[...] = it_fb

    iN = lax.broadcasted_iota(jnp.int32, (8, N), 1)
    d2o = d2_ref[...]
    sel = (d2o < t_ref[...]) | ((d2o == t_ref[...]) & (iN <= it_ref[...]))
    A = jnp.where(sel, 1.0 / KNN, 0.0).astype(jnp.bfloat16)
    agg = _dot_b(A, xm_ref[...])
    f = jnp.maximum(_dot_b(agg.astype(jnp.bfloat16), we_ref[...])
                    + be_ref[0:1, :], 0.0)
    fc1_ref[...] = _dot_b(f.astype(jnp.bfloat16), w1_ref[...])


def _run_knn(pc, pt2, ptb, xm, W_enc, b_enc, w1):
    return pl.pallas_call(
        _knn_body,
        grid=(COARSE // 8,),
        in_specs=[
            pl.BlockSpec((8, 3), lambda i: (i, 0)),
            pl.BlockSpec((3, N), lambda i: (0, 0)),
            pl.BlockSpec((3, N), lambda i: (0, 0)),
            pl.BlockSpec((N, 6), lambda i: (0, 0)),
            pl.BlockSpec((6, 512), lambda i: (0, 0)),
            pl.BlockSpec((1, 512), lambda i: (0, 0)),
            pl.BlockSpec((512, 256), lambda i: (0, 0)),
        ],
        out_specs=pl.BlockSpec((8, 256), lambda i: (i, 0)),
        out_shape=jax.ShapeDtypeStruct((COARSE, 256), jnp.float32),
        scratch_shapes=[pltpu.VMEM((8, N), jnp.float32),
                        pltpu.VMEM((8, G, GL), jnp.float32),
                        pltpu.VMEM((8, 1), jnp.float32),
                        pltpu.VMEM((8, 1), jnp.int32),
                        pltpu.VMEM((1, N), jnp.float32)],
    )(pc, pt2, ptb, xm, W_enc, b_enc, w1)


def _interp_body(p_ref, pcT_ref, fc1_ref, b1_ref, z1_ref, s1_ref, q1_ref):
    @pl.when(pl.program_id(0) == 0)
    def _():
        s1_ref[...] = jnp.zeros_like(s1_ref)
        q1_ref[...] = jnp.zeros_like(q1_ref)

    p = p_ref[...]
    px, py, pz = p[:, 0:1], p[:, 1:2], p[:, 2:3]
    sp = (px * px + py * py) + pz * pz
    cx = pcT_ref[0:1, :]
    cy = pcT_ref[1:2, :]
    cz = pcT_ref[2:3, :]
    sc = (cx * cx + cy * cy) + cz * cz
    dot = _dot_bf(p, pcT_ref[...])
    d2 = (sp + sc) - 2.0 * dot

    d0 = d2
    ds_ = []
    for j in range(3):
        rmin = jnp.min(d2, axis=1, keepdims=True)
        ds_.append(rmin)
        if j < 2:
            d2 = jnp.where(d2 == rmin, F_INF, d2)

    w = [1.0 / (jnp.sqrt(jnp.maximum(d, 1e-12)) + 1e-8) for d in ds_]
    wsum = (w[0] + w[1]) + w[2]
    wmat = (jnp.where(d0 == ds_[0], w[0] / wsum, 0.0)
            + jnp.where(d0 == ds_[1], w[1] / wsum, 0.0)
            + jnp.where(d0 == ds_[2], w[2] / wsum, 0.0))
    z1 = _dot_b(wmat.astype(jnp.bfloat16), fc1_ref[...]) + b1_ref[0:1, :]
    z1_ref[...] = z1
    s1_ref[...] += jnp.sum(z1, axis=0, keepdims=True)
    q1_ref[...] += jnp.sum(z1 * z1, axis=0, keepdims=True)


def _run_interp(p0, pcT, fc1, b1):
    return pl.pallas_call(
        _interp_body,
        grid=(NBLK,),
        in_specs=[
            pl.BlockSpec((BLK, 3), lambda i: (i, 0)),
            pl.BlockSpec((3, COARSE), lambda i: (0, 0)),
            pl.BlockSpec((COARSE, 256), lambda i: (0, 0)),
            pl.BlockSpec((1, 256), lambda i: (0, 0)),
        ],
        out_specs=[
            pl.BlockSpec((BLK, 256), lambda i: (i, 0)),
            pl.BlockSpec((1, 256), lambda i: (0, 0)),
            pl.BlockSpec((1, 256), lambda i: (0, 0)),
        ],
        out_shape=[
            jax.ShapeDtypeStruct((N, 256), jnp.float32),
            jax.ShapeDtypeStruct((1, 256), jnp.float32),
            jax.ShapeDtypeStruct((1, 256), jnp.float32),
        ],
    )(p0, pcT, fc1, b1)


def _bn_mm_body(z_ref, s_ref, q_ref, g_ref, be_ref, w_ref, bb_ref,
                o_ref, so_ref, qo_ref):
    @pl.when(pl.program_id(0) == 0)
    def _():
        so_ref[...] = jnp.zeros_like(so_ref)
        qo_ref[...] = jnp.zeros_like(qo_ref)

    inv_n = jnp.float32(1.0 / N)
    mu = s_ref[...] * inv_n
    var = q_ref[...] * inv_n - mu * mu
    z = z_ref[...]
    h = (z - mu) / jnp.sqrt(var + 1e-5) * g_ref[0:1, :] + be_ref[0:1, :]
    h = jnp.maximum(h, 0.0)
    o = _dot_b(h.astype(jnp.bfloat16), w_ref[...]) + bb_ref[0:1, :]
    o_ref[...] = o
    so_ref[...] += jnp.sum(o, axis=0, keepdims=True)
    qo_ref[...] += jnp.sum(o * o, axis=0, keepdims=True)


def _run_bn_mm(z, s, q, g, beta, w, b, din, dout):
    return pl.pallas_call(
        _bn_mm_body,
        grid=(NBLK,),
        in_specs=[
            pl.BlockSpec((BLK, din), lambda i: (i, 0)),
            pl.BlockSpec((1, din), lambda i: (0, 0)),
            pl.BlockSpec((1, din), lambda i: (0, 0)),
            pl.BlockSpec((1, din), lambda i: (0, 0)),
            pl.BlockSpec((1, din), lambda i: (0, 0)),
            pl.BlockSpec((din, dout), lambda i: (0, 0)),
            pl.BlockSpec((1, dout), lambda i: (0, 0)),
        ],
        out_specs=[
            pl.BlockSpec((BLK, dout), lambda i: (i, 0)),
            pl.BlockSpec((1, dout), lambda i: (0, 0)),
            pl.BlockSpec((1, dout), lambda i: (0, 0)),
        ],
        out_shape=[
            jax.ShapeDtypeStruct((N, dout), jnp.float32),
            jax.ShapeDtypeStruct((1, dout), jnp.float32),
            jax.ShapeDtypeStruct((1, dout), jnp.float32),
        ],
    )(z, s, q, g, beta, w, b)


def _final_body(z_ref, s_ref, q_ref, g_ref, be_ref, w_ref, bb_ref,
                xm_ref, mk_ref, xr_ref, ls_ref, ms_ref):
    @pl.when(pl.program_id(0) == 0)
    def _():
        ls_ref[...] = jnp.zeros_like(ls_ref)
        ms_ref[...] = jnp.zeros_like(ms_ref)

    inv_n = jnp.float32(1.0 / N)
    mu = s_ref[...] * inv_n
    var = q_ref[...] * inv_n - mu * mu
    h = (z_ref[...] - mu) / jnp.sqrt(var + 1e-5) * g_ref[0:1, :] + be_ref[0:1, :]
    h = jnp.maximum(h, 0.0)
    xr = _dot_b(h.astype(jnp.bfloat16), w_ref[...]) + bb_ref[0:1, :]
    xm = xm_ref[...]
    mk = mk_ref[...]
    l1 = jnp.abs(xr - xm) * mk
    ls_ref[...] += jnp.sum(l1, axis=(0, 1), keepdims=True)
    ms_ref[...] += jnp.sum(mk, axis=(0, 1), keepdims=True)
    xr_ref[...] = jnp.where(mk > 0.0, xr, xm)


def _run_final(z2, s2, q2, g2, beta2, w3, b3, xm, mk):
    return pl.pallas_call(
        _final_body,
        grid=(NBLK,),
        in_specs=[
            pl.BlockSpec((BLK, 128), lambda i: (i, 0)),
            pl.BlockSpec((1, 128), lambda i: (0, 0)),
            pl.BlockSpec((1, 128), lambda i: (0, 0)),
            pl.BlockSpec((1, 128), lambda i: (0, 0)),
            pl.BlockSpec((1, 128), lambda i: (0, 0)),
            pl.BlockSpec((128, 6), lambda i: (0, 0)),
            pl.BlockSpec((1, 6), lambda i: (0, 0)),
            pl.BlockSpec((BLK, 6), lambda i: (i, 0)),
            pl.BlockSpec((BLK, 1), lambda i: (i, 0)),
        ],
        out_specs=[
            pl.BlockSpec((BLK, 6), lambda i: (i, 0)),
            pl.BlockSpec((1, 1), lambda i: (0, 0)),
            pl.BlockSpec((1, 1), lambda i: (0, 0)),
        ],
        out_shape=[
            jax.ShapeDtypeStruct((N, 6), jnp.float32),
            jax.ShapeDtypeStruct((1, 1), jnp.float32),
            jax.ShapeDtypeStruct((1, 1), jnp.float32),
        ],
    )(z2, s2, q2, g2, beta2, w3, b3, xm, mk)


def kernel(point, features, W_enc, b_enc, w1, b1, g1, beta1, w2, b2, g2,
           beta2, w3, b3):
    p0 = point.reshape(-1, 3)
    x0 = features.reshape(-1, 6)
    pt2 = p0.T
    pt3 = pt2.reshape(3, SUB, LANES)

    masked_clusters = jax.random.permutation(
        jax.random.key(1), M)[:NUM_MASK].astype(jnp.int32)
    flags = jnp.isin(jnp.arange(M, dtype=jnp.int32),
                     masked_clusters).astype(jnp.float32).reshape(1, M)

    pc = _run_fps(pt3, p0)
    pcT = pc.T

    bf = jnp.bfloat16
    xm, mk = _run_cluster(p0, x0, pcT, flags)
    fc1 = _run_knn(pc, pt2, pt2.astype(bf), xm.astype(bf), W_enc.astype(bf),
                   b_enc.reshape(1, -1), w1.astype(bf))
    z1, s1, q1 = _run_interp(p0, pcT, fc1.astype(bf), b1.reshape(1, -1))
    z2, s2, q2 = _run_bn_mm(z1, s1, q1, g1.reshape(1, -1), beta1.reshape(1, -1),
                            w2.astype(bf), b2.reshape(1, -1), 256, 128)
    xrec, lsum, msum = _run_final(z2, s2, q2, g2.reshape(1, -1),
                                  beta2.reshape(1, -1), w3.astype(bf),
                                  b3.reshape(1, -1), xm, mk)

    loss = lsum[0, 0] / (msum[0, 0] * 6.0)
    mask = mk[:, 0] > 0.5
    visible = jnp.logical_not(mask)
    return (loss, xrec, mask, visible, xm)


# speedup vs baseline: 1.6614x; 1.1576x over previous
import functools

import jax
import jax.numpy as jnp
from jax import lax
from jax.experimental import pallas as pl
from jax.experimental.pallas import tpu as pltpu

N = 24000
M = 64
COARSE = 512
NUM_MASK = 19
KNN = 16
SUB = 8
LANES = N // SUB
BLK = 480
NBLK = N // BLK
F_INF = 3e38
IBIG = 1 << 30

_HIGH = jax.lax.Precision.HIGHEST


def _dot(a, b):
    return jax.lax.dot_general(
        a, b, (((1,), (0,)), ((), ())),
        precision=_HIGH, preferred_element_type=jnp.float32)


def _dot_b(a, b):
    return jax.lax.dot_general(
        a, b, (((1,), (0,)), ((), ())), preferred_element_type=jnp.float32)


def _dot_bf(a, b):
    return jax.lax.dot_general(
        a.astype(jnp.bfloat16), b.astype(jnp.bfloat16), (((1,), (0,)), ((), ())),
        preferred_element_type=jnp.float32)


def _fps_body(pt_ref, p0_ref, pc_ref, dist_ref):
    dist_ref[...] = jnp.full((SUB, LANES), 1e10, jnp.float32)
    ii = (lax.broadcasted_iota(jnp.int32, (SUB, LANES), 0) * LANES
          + lax.broadcasted_iota(jnp.int32, (SUB, LANES), 1))

    def body(k, last):
        row = p0_ref[pl.ds(last, 1), :]
        pc_ref[pl.ds(k, 1), :] = row
        cx, cy, cz = row[:, 0:1], row[:, 1:2], row[:, 2:3]
        dx = pt_ref[0] - cx
        dy = pt_ref[1] - cy
        dz = pt_ref[2] - cz
        d = (dx * dx + dy * dy) + dz * dz
        dist = jnp.minimum(dist_ref[...], d)
        dist_ref[...] = dist
        m = jnp.max(dist, axis=(0, 1), keepdims=True)
        nxt = jnp.min(jnp.where(dist == m, ii, IBIG))
        return nxt

    last = lax.fori_loop(0, COARSE - 1, body, jnp.int32(0), unroll=2)
    pc_ref[pl.ds(COARSE - 1, 1), :] = p0_ref[pl.ds(last, 1), :]


def _run_fps(pt3, p0):
    return pl.pallas_call(
        _fps_body,
        out_shape=jax.ShapeDtypeStruct((COARSE, 3), jnp.float32),
        scratch_shapes=[pltpu.VMEM((SUB, LANES), jnp.float32)],
    )(pt3, p0)


def _cluster_body(p_ref, x_ref, pcT_ref, fl_ref, xm_ref, mk_ref):
    p = p_ref[...]
    px, py, pz = p[:, 0:1], p[:, 1:2], p[:, 2:3]
    sp = (px * px + py * py) + pz * pz
    cx = pcT_ref[0:1, 0:M]
    cy = pcT_ref[1:2, 0:M]
    cz = pcT_ref[2:3, 0:M]
    sc = (cx * cx + cy * cy) + cz * cz
    dot = _dot_bf(p, pcT_ref[:, 0:M])
    d2 = (sp + sc) - 2.0 * dot
    rmin = jnp.min(d2, axis=1, keepdims=True)
    i64 = lax.broadcasted_iota(jnp.int32, d2.shape, 1)
    cl = jnp.min(jnp.where(d2 == rmin, i64, IBIG), axis=1, keepdims=True)
    onehot = i64 == cl
    maskf = jnp.max(jnp.where(onehot, fl_ref[0:1, :], 0.0), axis=1,
                    keepdims=True)
    mk_ref[...] = maskf
    xm_ref[...] = jnp.where(maskf > 0.0, 0.0, x_ref[...])


def _run_cluster(p0, x0, pcT, flags):
    blk = 3000
    grid = N // blk
    return pl.pallas_call(
        _cluster_body,
        grid=(grid,),
        in_specs=[
            pl.BlockSpec((blk, 3), lambda i: (i, 0)),
            pl.BlockSpec((blk, 6), lambda i: (i, 0)),
            pl.BlockSpec((3, COARSE), lambda i: (0, 0)),
            pl.BlockSpec((1, M), lambda i: (0, 0)),
        ],
        out_specs=[
            pl.BlockSpec((blk, 6), lambda i: (i, 0)),
            pl.BlockSpec((blk, 1), lambda i: (i, 0)),
        ],
        out_shape=[
            jax.ShapeDtypeStruct((N, 6), jnp.float32),
            jax.ShapeDtypeStruct((N, 1), jnp.float32),
        ],
    )(p0, x0, pcT, flags)


G = 64
GL = N // G
RB = 16
TOPG = 4


def _knn_body(pc_ref, pt_ref, ptb_ref, xm_ref, we_ref, be_ref, w1_ref, fc1_ref,
              d2_ref, x_ref, t_ref, it_ref, sp_ref):
    @pl.when(pl.program_id(0) == 0)
    def _():
        px = pt_ref[0:1, :]
        py = pt_ref[1:2, :]
        pz = pt_ref[2:3, :]
        sp_ref[...] = (px * px + py * py) + pz * pz

    c = pc_ref[...]
    cx, cy, cz = c[:, 0:1], c[:, 1:2], c[:, 2:3]
    sc = (cx * cx + cy * cy) + cz * cz
    dot = jax.lax.dot_general(
        c.astype(jnp.bfloat16), ptb_ref[...], (((1,), (0,)), ((), ())),
        preferred_element_type=jnp.float32)
    d2 = (sc + sp_ref[...]) - 2.0 * dot
    d2_ref[...] = d2

    x_ref[...] = d2.reshape(RB, G, GL)
    cands = []
    for j in range(TOPG):
        x = x_ref[...]
        m = jnp.min(x, axis=2, keepdims=True)
        if j + 1 < TOPG:
            x_ref[...] = jnp.where(x == m, F_INF, x)
        cands.append(m)
    cand = jnp.concatenate(cands, axis=2).reshape(RB, G * TOPG)

    t = cand
    for j in range(KNN):
        m = jnp.min(cand, axis=1, keepdims=True)
        if j + 1 < KNN:
            cand = jnp.where(cand == m, F_INF, cand)
        t = m
    t_ref[...] = t
    it_ref[...] = jnp.full((RB, 1), IBIG - 1, jnp.int32)

    cnt = jnp.sum((d2 <= t).astype(jnp.float32), axis=1, keepdims=True)
    bad = jnp.max(jnp.abs(cnt - KNN)) > 0.0

    @pl.when(bad)
    def _():
        x_ref[...] = d2_ref[...].reshape(RB, G, GL)
        ig = (lax.broadcasted_iota(jnp.int32, (RB, G, GL), 1) * GL
              + lax.broadcasted_iota(jnp.int32, (RB, G, GL), 2))

        def fb_body(_, carry):
            x = x_ref[...]
            rmin = jnp.min(x, axis=(1, 2), keepdims=True)
            imin = jnp.min(jnp.where(x == rmin, ig, IBIG), axis=(1, 2),
                           keepdims=True)
            x_ref[...] = jnp.where(ig == imin, F_INF, x)
            return rmin[:, 0, :], imin[:, 0, :]
        t_fb, it_fb = lax.fori_loop(
            0, KNN, fb_body,
            (jnp.zeros((RB, 1), jnp.float32), jnp.zeros((RB, 1), jnp.int32)))
        t_ref[...] = t_fb
        it_ref[...] = it_fb

    iN = lax.broadcasted_iota(jnp.int32, (RB, N), 1)
    d2o = d2_ref[...]
    sel = (d2o < t_ref[...]) | ((d2o == t_ref[...]) & (iN <= it_ref[...]))
    A = jnp.where(sel, 1.0 / KNN, 0.0).astype(jnp.bfloat16)
    agg = _dot_b(A, xm_ref[...])
    f = jnp.maximum(_dot_b(agg.astype(jnp.bfloat16), we_ref[...])
                    + be_ref[0:1, :], 0.0)
    fc1_ref[...] = _dot_b(f.astype(jnp.bfloat16), w1_ref[...])


def _run_knn(pc, pt2, ptb, xm, W_enc, b_enc, w1):
    return pl.pallas_call(
        _knn_body,
        grid=(COARSE // RB,),
        in_specs=[
            pl.BlockSpec((RB, 3), lambda i: (i, 0)),
            pl.BlockSpec((3, N), lambda i: (0, 0)),
            pl.BlockSpec((3, N), lambda i: (0, 0)),
            pl.BlockSpec((N, 6), lambda i: (0, 0)),
            pl.BlockSpec((6, 512), lambda i: (0, 0)),
            pl.BlockSpec((1, 512), lambda i: (0, 0)),
            pl.BlockSpec((512, 256), lambda i: (0, 0)),
        ],
        out_specs=pl.BlockSpec((RB, 256), lambda i: (i, 0)),
        out_shape=jax.ShapeDtypeStruct((COARSE, 256), jnp.float32),
        scratch_shapes=[pltpu.VMEM((RB, N), jnp.float32),
                        pltpu.VMEM((RB, G, GL), jnp.float32),
                        pltpu.VMEM((RB, 1), jnp.float32),
                        pltpu.VMEM((RB, 1), jnp.int32),
                        pltpu.VMEM((1, N), jnp.float32)],
    )(pc, pt2, ptb, xm, W_enc, b_enc, w1)


def _interp_body(p_ref, pcT_ref, fc1_ref, b1_ref, z1_ref, s1_ref, q1_ref):
    @pl.when(pl.program_id(0) == 0)
    def _():
        s1_ref[...] = jnp.zeros_like(s1_ref)
        q1_ref[...] = jnp.zeros_like(q1_ref)

    p = p_ref[...]
    px, py, pz = p[:, 0:1], p[:, 1:2], p[:, 2:3]
    sp = (px * px + py * py) + pz * pz
    cx = pcT_ref[0:1, :]
    cy = pcT_ref[1:2, :]
    cz = pcT_ref[2:3, :]
    sc = (cx * cx + cy * cy) + cz * cz
    dot = _dot_bf(p, pcT_ref[...])
    d2 = (sp + sc) - 2.0 * dot

    d0 = d2
    ds_ = []
    for j in range(3):
        rmin = jnp.min(d2, axis=1, keepdims=True)
        ds_.append(rmin)
        if j < 2:
            d2 = jnp.where(d2 == rmin, F_INF, d2)

    w = [1.0 / (jnp.sqrt(jnp.maximum(d, 1e-12)) + 1e-8) for d in ds_]
    wsum = (w[0] + w[1]) + w[2]
    wmat = (jnp.where(d0 == ds_[0], w[0] / wsum, 0.0)
            + jnp.where(d0 == ds_[1], w[1] / wsum, 0.0)
            + jnp.where(d0 == ds_[2], w[2] / wsum, 0.0))
    z1 = _dot_b(wmat.astype(jnp.bfloat16), fc1_ref[...]) + b1_ref[0:1, :]
    z1_ref[...] = z1
    s1_ref[...] += jnp.sum(z1, axis=0, keepdims=True)
    q1_ref[...] += jnp.sum(z1 * z1, axis=0, keepdims=True)


def _run_interp(p0, pcT, fc1, b1):
    return pl.pallas_call(
        _interp_body,
        grid=(NBLK,),
        in_specs=[
            pl.BlockSpec((BLK, 3), lambda i: (i, 0)),
            pl.BlockSpec((3, COARSE), lambda i: (0, 0)),
            pl.BlockSpec((COARSE, 256), lambda i: (0, 0)),
            pl.BlockSpec((1, 256), lambda i: (0, 0)),
        ],
        out_specs=[
            pl.BlockSpec((BLK, 256), lambda i: (i, 0)),
            pl.BlockSpec((1, 256), lambda i: (0, 0)),
            pl.BlockSpec((1, 256), lambda i: (0, 0)),
        ],
        out_shape=[
            jax.ShapeDtypeStruct((N, 256), jnp.float32),
            jax.ShapeDtypeStruct((1, 256), jnp.float32),
            jax.ShapeDtypeStruct((1, 256), jnp.float32),
        ],
    )(p0, pcT, fc1, b1)


def _bn_mm_body(z_ref, s_ref, q_ref, g_ref, be_ref, w_ref, bb_ref,
                o_ref, so_ref, qo_ref):
    @pl.when(pl.program_id(0) == 0)
    def _():
        so_ref[...] = jnp.zeros_like(so_ref)
        qo_ref[...] = jnp.zeros_like(qo_ref)

    inv_n = jnp.float32(1.0 / N)
    mu = s_ref[...] * inv_n
    var = q_ref[...] * inv_n - mu * mu
    z = z_ref[...]
    h = (z - mu) / jnp.sqrt(var + 1e-5) * g_ref[0:1, :] + be_ref[0:1, :]
    h = jnp.maximum(h, 0.0)
    o = _dot_b(h.astype(jnp.bfloat16), w_ref[...]) + bb_ref[0:1, :]
    o_ref[...] = o
    so_ref[...] += jnp.sum(o, axis=0, keepdims=True)
    qo_ref[...] += jnp.sum(o * o, axis=0, keepdims=True)


def _run_bn_mm(z, s, q, g, beta, w, b, din, dout):
    return pl.pallas_call(
        _bn_mm_body,
        grid=(NBLK,),
        in_specs=[
            pl.BlockSpec((BLK, din), lambda i: (i, 0)),
            pl.BlockSpec((1, din), lambda i: (0, 0)),
            pl.BlockSpec((1, din), lambda i: (0, 0)),
            pl.BlockSpec((1, din), lambda i: (0, 0)),
            pl.BlockSpec((1, din), lambda i: (0, 0)),
            pl.BlockSpec((din, dout), lambda i: (0, 0)),
            pl.BlockSpec((1, dout), lambda i: (0, 0)),
        ],
        out_specs=[
            pl.BlockSpec((BLK, dout), lambda i: (i, 0)),
            pl.BlockSpec((1, dout), lambda i: (0, 0)),
            pl.BlockSpec((1, dout), lambda i: (0, 0)),
        ],
        out_shape=[
            jax.ShapeDtypeStruct((N, dout), jnp.float32),
            jax.ShapeDtypeStruct((1, dout), jnp.float32),
            jax.ShapeDtypeStruct((1, dout), jnp.float32),
        ],
    )(z, s, q, g, beta, w, b)


def _final_body(z_ref, s_ref, q_ref, g_ref, be_ref, w_ref, bb_ref,
                xm_ref, mk_ref, xr_ref, ls_ref, ms_ref):
    @pl.when(pl.program_id(0) == 0)
    def _():
        ls_ref[...] = jnp.zeros_like(ls_ref)
        ms_ref[...] = jnp.zeros_like(ms_ref)

    inv_n = jnp.float32(1.0 / N)
    mu = s_ref[...] * inv_n
    var = q_ref[...] * inv_n - mu * mu
    h = (z_ref[...] - mu) / jnp.sqrt(var + 1e-5) * g_ref[0:1, :] + be_ref[0:1, :]
    h = jnp.maximum(h, 0.0)
    xr = _dot_b(h.astype(jnp.bfloat16), w_ref[...]) + bb_ref[0:1, :]
    xm = xm_ref[...]
    mk = mk_ref[...]
    l1 = jnp.abs(xr - xm) * mk
    ls_ref[...] += jnp.sum(l1, axis=(0, 1), keepdims=True)
    ms_ref[...] += jnp.sum(mk, axis=(0, 1), keepdims=True)
    xr_ref[...] = jnp.where(mk > 0.0, xr, xm)


def _run_final(z2, s2, q2, g2, beta2, w3, b3, xm, mk):
    return pl.pallas_call(
        _final_body,
        grid=(NBLK,),
        in_specs=[
            pl.BlockSpec((BLK, 128), lambda i: (i, 0)),
            pl.BlockSpec((1, 128), lambda i: (0, 0)),
            pl.BlockSpec((1, 128), lambda i: (0, 0)),
            pl.BlockSpec((1, 128), lambda i: (0, 0)),
            pl.BlockSpec((1, 128), lambda i: (0, 0)),
            pl.BlockSpec((128, 6), lambda i: (0, 0)),
            pl.BlockSpec((1, 6), lambda i: (0, 0)),
            pl.BlockSpec((BLK, 6), lambda i: (i, 0)),
            pl.BlockSpec((BLK, 1), lambda i: (i, 0)),
        ],
        out_specs=[
            pl.BlockSpec((BLK, 6), lambda i: (i, 0)),
            pl.BlockSpec((1, 1), lambda i: (0, 0)),
            pl.BlockSpec((1, 1), lambda i: (0, 0)),
        ],
        out_shape=[
            jax.ShapeDtypeStruct((N, 6), jnp.float32),
            jax.ShapeDtypeStruct((1, 1), jnp.float32),
            jax.ShapeDtypeStruct((1, 1), jnp.float32),
        ],
    )(z2, s2, q2, g2, beta2, w3, b3, xm, mk)


def kernel(point, features, W_enc, b_enc, w1, b1, g1, beta1, w2, b2, g2,
           beta2, w3, b3):
    p0 = point.reshape(-1, 3)
    x0 = features.reshape(-1, 6)
    pt2 = p0.T
    pt3 = pt2.reshape(3, SUB, LANES)

    masked_clusters = jax.random.permutation(
        jax.random.key(1), M)[:NUM_MASK].astype(jnp.int32)
    flags = jnp.isin(jnp.arange(M, dtype=jnp.int32),
                     masked_clusters).astype(jnp.float32).reshape(1, M)

    pc = _run_fps(pt3, p0)
    pcT = pc.T

    bf = jnp.bfloat16
    xm, mk = _run_cluster(p0, x0, pcT, flags)
    fc1 = _run_knn(pc, pt2, pt2.astype(bf), xm.astype(bf), W_enc.astype(bf),
                   b_enc.reshape(1, -1), w1.astype(bf))
    z1, s1, q1 = _run_interp(p0, pcT, fc1.astype(bf), b1.reshape(1, -1))
    z2, s2, q2 = _run_bn_mm(z1, s1, q1, g1.reshape(1, -1), beta1.reshape(1, -1),
                            w2.astype(bf), b2.reshape(1, -1), 256, 128)
    xrec, lsum, msum = _run_final(z2, s2, q2, g2.reshape(1, -1),
                                  beta2.reshape(1, -1), w3.astype(bf),
                                  b3.reshape(1, -1), xm, mk)

    loss = lsum[0, 0] / (msum[0, 0] * 6.0)
    mask = mk[:, 0] > 0.5
    visible = jnp.logical_not(mask)
    return (loss, xrec, mask, visible, xm)


# speedup vs baseline: 1.6617x; 1.0002x over previous
import functools

import jax
import jax.numpy as jnp
from jax import lax
from jax.experimental import pallas as pl
from jax.experimental.pallas import tpu as pltpu

N = 24000
M = 64
COARSE = 512
NUM_MASK = 19
KNN = 16
SUB = 8
LANES = N // SUB
BLK = 480
NBLK = N // BLK
F_INF = 3e38
IBIG = 1 << 30

_HIGH = jax.lax.Precision.HIGHEST


def _dot(a, b):
    return jax.lax.dot_general(
        a, b, (((1,), (0,)), ((), ())),
        precision=_HIGH, preferred_element_type=jnp.float32)


def _dot_b(a, b):
    return jax.lax.dot_general(
        a, b, (((1,), (0,)), ((), ())), preferred_element_type=jnp.float32)


def _dot_bf(a, b):
    return jax.lax.dot_general(
        a.astype(jnp.bfloat16), b.astype(jnp.bfloat16), (((1,), (0,)), ((), ())),
        preferred_element_type=jnp.float32)


def _fps_body(pt_ref, p0_ref, pc_ref, dist_ref):
    dist_ref[...] = jnp.full((SUB, LANES), 1e10, jnp.float32)
    ii = (lax.broadcasted_iota(jnp.int32, (SUB, LANES), 0) * LANES
          + lax.broadcasted_iota(jnp.int32, (SUB, LANES), 1))

    def body(k, last):
        row = p0_ref[pl.ds(last, 1), :]
        pc_ref[pl.ds(k, 1), :] = row
        cx, cy, cz = row[:, 0:1], row[:, 1:2], row[:, 2:3]
        dx = pt_ref[0] - cx
        dy = pt_ref[1] - cy
        dz = pt_ref[2] - cz
        d = (dx * dx + dy * dy) + dz * dz
        dist = jnp.minimum(dist_ref[...], d)
        dist_ref[...] = dist
        m = jnp.max(dist, axis=(0, 1), keepdims=True)
        nxt = jnp.min(jnp.where(dist == m, ii, IBIG))
        return nxt

    last = lax.fori_loop(0, COARSE - 1, body, jnp.int32(0), unroll=2)
    pc_ref[pl.ds(COARSE - 1, 1), :] = p0_ref[pl.ds(last, 1), :]


def _run_fps(pt3, p0):
    return pl.pallas_call(
        _fps_body,
        out_shape=jax.ShapeDtypeStruct((COARSE, 3), jnp.float32),
        scratch_shapes=[pltpu.VMEM((SUB, LANES), jnp.float32)],
    )(pt3, p0)


def _cluster_body(p_ref, x_ref, pcT_ref, fl_ref, xm_ref, mk_ref):
    p = p_ref[...]
    px, py, pz = p[:, 0:1], p[:, 1:2], p[:, 2:3]
    sp = (px * px + py * py) + pz * pz
    cx = pcT_ref[0:1, 0:M]
    cy = pcT_ref[1:2, 0:M]
    cz = pcT_ref[2:3, 0:M]
    sc = (cx * cx + cy * cy) + cz * cz
    dot = _dot_bf(p, pcT_ref[:, 0:M])
    d2 = (sp + sc) - 2.0 * dot
    rmin = jnp.min(d2, axis=1, keepdims=True)
    i64 = lax.broadcasted_iota(jnp.int32, d2.shape, 1)
    cl = jnp.min(jnp.where(d2 == rmin, i64, IBIG), axis=1, keepdims=True)
    onehot = i64 == cl
    maskf = jnp.max(jnp.where(onehot, fl_ref[0:1, :], 0.0), axis=1,
                    keepdims=True)
    mk_ref[...] = maskf
    xm_ref[...] = jnp.where(maskf > 0.0, 0.0, x_ref[...])


def _run_cluster(p0, x0, pcT, flags):
    blk = 3000
    grid = N // blk
    return pl.pallas_call(
        _cluster_body,
        grid=(grid,),
        in_specs=[
            pl.BlockSpec((blk, 3), lambda i: (i, 0)),
            pl.BlockSpec((blk, 6), lambda i: (i, 0)),
            pl.BlockSpec((3, COARSE), lambda i: (0, 0)),
            pl.BlockSpec((1, M), lambda i: (0, 0)),
        ],
        out_specs=[
            pl.BlockSpec((blk, 6), lambda i: (i, 0)),
            pl.BlockSpec((blk, 1), lambda i: (i, 0)),
        ],
        out_shape=[
            jax.ShapeDtypeStruct((N, 6), jnp.float32),
            jax.ShapeDtypeStruct((N, 1), jnp.float32),
        ],
    )(p0, x0, pcT, flags)


G = 64
GL = N // G
RB = 16
TOPG = 4


def _knn_body(pc_ref, pt_ref, ptb_ref, xm_ref, we_ref, be_ref, w1_ref, fc1_ref,
              d2_ref, x_ref, t_ref, it_ref, sp_ref):
    @pl.when(pl.program_id(0) == 0)
    def _():
        px = pt_ref[0:1, :]
        py = pt_ref[1:2, :]
        pz = pt_ref[2:3, :]
        sp_ref[...] = (px * px + py * py) + pz * pz

    c = pc_ref[...]
    cx, cy, cz = c[:, 0:1], c[:, 1:2], c[:, 2:3]
    sc = (cx * cx + cy * cy) + cz * cz
    dot = jax.lax.dot_general(
        c.astype(jnp.bfloat16), ptb_ref[...], (((1,), (0,)), ((), ())),
        preferred_element_type=jnp.float32)
    d2 = (sc + sp_ref[...]) - 2.0 * dot
    d2_ref[...] = d2

    x_ref[...] = d2.reshape(RB, G, GL)
    cands = []
    for j in range(TOPG):
        x = x_ref[...]
        m = jnp.min(x, axis=2, keepdims=True)
        if j + 1 < TOPG:
            x_ref[...] = jnp.where(x == m, F_INF, x)
        cands.append(m)
    cand = jnp.concatenate(cands, axis=2).reshape(RB, G * TOPG)

    t = cand
    for j in range(KNN):
        m = jnp.min(cand, axis=1, keepdims=True)
        if j + 1 < KNN:
            cand = jnp.where(cand == m, F_INF, cand)
        t = m
    t_ref[...] = t
    it_ref[...] = jnp.full((RB, 1), IBIG - 1, jnp.int32)

    cnt = jnp.sum((d2 <= t).astype(jnp.float32), axis=1, keepdims=True)
    bad = jnp.max(jnp.abs(cnt - KNN)) > 0.0

    @pl.when(bad)
    def _():
        x_ref[...] = d2_ref[...].reshape(RB, G, GL)
        ig = (lax.broadcasted_iota(jnp.int32, (RB, G, GL), 1) * GL
              + lax.broadcasted_iota(jnp.int32, (RB, G, GL), 2))

        def fb_body(_, carry):
            x = x_ref[...]
            rmin = jnp.min(x, axis=(1, 2), keepdims=True)
            imin = jnp.min(jnp.where(x == rmin, ig, IBIG), axis=(1, 2),
                           keepdims=True)
            x_ref[...] = jnp.where(ig == imin, F_INF, x)
            return rmin[:, 0, :], imin[:, 0, :]
        t_fb, it_fb = lax.fori_loop(
            0, KNN, fb_body,
            (jnp.zeros((RB, 1), jnp.float32), jnp.zeros((RB, 1), jnp.int32)))
        t_ref[...] = t_fb
        it_ref[...] = it_fb

    iN = lax.broadcasted_iota(jnp.int32, (RB, N), 1)
    d2o = d2_ref[...]
    sel = (d2o < t_ref[...]) | ((d2o == t_ref[...]) & (iN <= it_ref[...]))
    A = jnp.where(sel, 1.0 / KNN, 0.0).astype(jnp.bfloat16)
    agg = _dot_b(A, xm_ref[...])
    f = jnp.maximum(_dot_b(agg.astype(jnp.bfloat16), we_ref[...])
                    + be_ref[0:1, :], 0.0)
    fc1_ref[...] = _dot_b(f.astype(jnp.bfloat16), w1_ref[...])


def _run_knn(pc, pt2, ptb, xm, W_enc, b_enc, w1):
    return pl.pallas_call(
        _knn_body,
        grid=(COARSE // RB,),
        in_specs=[
            pl.BlockSpec((RB, 3), lambda i: (i, 0)),
            pl.BlockSpec((3, N), lambda i: (0, 0)),
            pl.BlockSpec((3, N), lambda i: (0, 0)),
            pl.BlockSpec((N, 6), lambda i: (0, 0)),
            pl.BlockSpec((6, 512), lambda i: (0, 0)),
            pl.BlockSpec((1, 512), lambda i: (0, 0)),
            pl.BlockSpec((512, 256), lambda i: (0, 0)),
        ],
        out_specs=pl.BlockSpec((RB, 256), lambda i: (i, 0)),
        out_shape=jax.ShapeDtypeStruct((COARSE, 256), jnp.float32),
        scratch_shapes=[pltpu.VMEM((RB, N), jnp.float32),
                        pltpu.VMEM((RB, G, GL), jnp.float32),
                        pltpu.VMEM((RB, 1), jnp.float32),
                        pltpu.VMEM((RB, 1), jnp.int32),
                        pltpu.VMEM((1, N), jnp.float32)],
    )(pc, pt2, ptb, xm, W_enc, b_enc, w1)


def _interp_body(p_ref, pcT_ref, fc1_ref, b1_ref, z1_ref, s1_ref, q1_ref):
    @pl.when(pl.program_id(0) == 0)
    def _():
        s1_ref[...] = jnp.zeros_like(s1_ref)
        q1_ref[...] = jnp.zeros_like(q1_ref)

    p = p_ref[...]
    px, py, pz = p[:, 0:1], p[:, 1:2], p[:, 2:3]
    sp = (px * px + py * py) + pz * pz
    cx = pcT_ref[0:1, :]
    cy = pcT_ref[1:2, :]
    cz = pcT_ref[2:3, :]
    sc = (cx * cx + cy * cy) + cz * cz
    dot = _dot_bf(p, pcT_ref[...])
    d2 = (sp + sc) - 2.0 * dot

    ii = lax.broadcasted_iota(jnp.int32, d2.shape, 1)
    ds_ = []
    is_ = []
    for j in range(3):
        rmin = jnp.min(d2, axis=1, keepdims=True)
        imin = jnp.min(jnp.where(d2 == rmin, ii, IBIG), axis=1, keepdims=True)
        ds_.append(rmin)
        is_.append(imin)
        if j < 2:
            d2 = jnp.where(ii == imin, F_INF, d2)

    w = [1.0 / (jnp.sqrt(jnp.maximum(d, 1e-12)) + 1e-8) for d in ds_]
    wsum = (w[0] + w[1]) + w[2]
    wmat = (jnp.where(ii == is_[0], w[0] / wsum, 0.0)
            + jnp.where(ii == is_[1], w[1] / wsum, 0.0)
            + jnp.where(ii == is_[2], w[2] / wsum, 0.0))
    z1 = _dot_b(wmat.astype(jnp.bfloat16), fc1_ref[...]) + b1_ref[0:1, :]
    z1_ref[...] = z1
    s1_ref[...] += jnp.sum(z1, axis=0, keepdims=True)
    q1_ref[...] += jnp.sum(z1 * z1, axis=0, keepdims=True)


def _run_interp(p0, pcT, fc1, b1):
    return pl.pallas_call(
        _interp_body,
        grid=(NBLK,),
        in_specs=[
            pl.BlockSpec((BLK, 3), lambda i: (i, 0)),
            pl.BlockSpec((3, COARSE), lambda i: (0, 0)),
            pl.BlockSpec((COARSE, 256), lambda i: (0, 0)),
            pl.BlockSpec((1, 256), lambda i: (0, 0)),
        ],
        out_specs=[
            pl.BlockSpec((BLK, 256), lambda i: (i, 0)),
            pl.BlockSpec((1, 256), lambda i: (0, 0)),
            pl.BlockSpec((1, 256), lambda i: (0, 0)),
        ],
        out_shape=[
            jax.ShapeDtypeStruct((N, 256), jnp.float32),
            jax.ShapeDtypeStruct((1, 256), jnp.float32),
            jax.ShapeDtypeStruct((1, 256), jnp.float32),
        ],
    )(p0, pcT, fc1, b1)


def _bn_mm_body(z_ref, s_ref, q_ref, g_ref, be_ref, w_ref, bb_ref,
                o_ref, so_ref, qo_ref):
    @pl.when(pl.program_id(0) == 0)
    def _():
        so_ref[...] = jnp.zeros_like(so_ref)
        qo_ref[...] = jnp.zeros_like(qo_ref)

    inv_n = jnp.float32(1.0 / N)
    mu = s_ref[...] * inv_n
    var = q_ref[...] * inv_n - mu * mu
    z = z_ref[...]
    h = (z - mu) / jnp.sqrt(var + 1e-5) * g_ref[0:1, :] + be_ref[0:1, :]
    h = jnp.maximum(h, 0.0)
    o = _dot_b(h.astype(jnp.bfloat16), w_ref[...]) + bb_ref[0:1, :]
    o_ref[...] = o
    so_ref[...] += jnp.sum(o, axis=0, keepdims=True)
    qo_ref[...] += jnp.sum(o * o, axis=0, keepdims=True)


def _run_bn_mm(z, s, q, g, beta, w, b, din, dout):
    return pl.pallas_call(
        _bn_mm_body,
        grid=(NBLK,),
        in_specs=[
            pl.BlockSpec((BLK, din), lambda i: (i, 0)),
            pl.BlockSpec((1, din), lambda i: (0, 0)),
            pl.BlockSpec((1, din), lambda i: (0, 0)),
            pl.BlockSpec((1, din), lambda i: (0, 0)),
            pl.BlockSpec((1, din), lambda i: (0, 0)),
            pl.BlockSpec((din, dout), lambda i: (0, 0)),
            pl.BlockSpec((1, dout), lambda i: (0, 0)),
        ],
        out_specs=[
            pl.BlockSpec((BLK, dout), lambda i: (i, 0)),
            pl.BlockSpec((1, dout), lambda i: (0, 0)),
            pl.BlockSpec((1, dout), lambda i: (0, 0)),
        ],
        out_shape=[
            jax.ShapeDtypeStruct((N, dout), jnp.float32),
            jax.ShapeDtypeStruct((1, dout), jnp.float32),
            jax.ShapeDtypeStruct((1, dout), jnp.float32),
        ],
    )(z, s, q, g, beta, w, b)


def _final_body(z_ref, s_ref, q_ref, g_ref, be_ref, w_ref, bb_ref,
                xm_ref, mk_ref, xr_ref, ls_ref, ms_ref):
    @pl.when(pl.program_id(0) == 0)
    def _():
        ls_ref[...] = jnp.zeros_like(ls_ref)
        ms_ref[...] = jnp.zeros_like(ms_ref)

    inv_n = jnp.float32(1.0 / N)
    mu = s_ref[...] * inv_n
    var = q_ref[...] * inv_n - mu * mu
    h = (z_ref[...] - mu) / jnp.sqrt(var + 1e-5) * g_ref[0:1, :] + be_ref[0:1, :]
    h = jnp.maximum(h, 0.0)
    xr = _dot_b(h.astype(jnp.bfloat16), w_ref[...]) + bb_ref[0:1, :]
    xm = xm_ref[...]
    mk = mk_ref[...]
    l1 = jnp.abs(xr - xm) * mk
    ls_ref[...] += jnp.sum(l1, axis=(0, 1), keepdims=True)
    ms_ref[...] += jnp.sum(mk, axis=(0, 1), keepdims=True)
    xr_ref[...] = jnp.where(mk > 0.0, xr, xm)


def _run_final(z2, s2, q2, g2, beta2, w3, b3, xm, mk):
    return pl.pallas_call(
        _final_body,
        grid=(NBLK,),
        in_specs=[
            pl.BlockSpec((BLK, 128), lambda i: (i, 0)),
            pl.BlockSpec((1, 128), lambda i: (0, 0)),
            pl.BlockSpec((1, 128), lambda i: (0, 0)),
            pl.BlockSpec((1, 128), lambda i: (0, 0)),
            pl.BlockSpec((1, 128), lambda i: (0, 0)),
            pl.BlockSpec((128, 6), lambda i: (0, 0)),
            pl.BlockSpec((1, 6), lambda i: (0, 0)),
            pl.BlockSpec((BLK, 6), lambda i: (i, 0)),
            pl.BlockSpec((BLK, 1), lambda i: (i, 0)),
        ],
        out_specs=[
            pl.BlockSpec((BLK, 6), lambda i: (i, 0)),
            pl.BlockSpec((1, 1), lambda i: (0, 0)),
            pl.BlockSpec((1, 1), lambda i: (0, 0)),
        ],
        out_shape=[
            jax.ShapeDtypeStruct((N, 6), jnp.float32),
            jax.ShapeDtypeStruct((1, 1), jnp.float32),
            jax.ShapeDtypeStruct((1, 1), jnp.float32),
        ],
    )(z2, s2, q2, g2, beta2, w3, b3, xm, mk)


def kernel(point, features, W_enc, b_enc, w1, b1, g1, beta1, w2, b2, g2,
           beta2, w3, b3):
    p0 = point.reshape(-1, 3)
    x0 = features.reshape(-1, 6)
    pt2 = p0.T
    pt3 = pt2.reshape(3, SUB, LANES)

    masked_clusters = jax.random.permutation(
        jax.random.key(1), M)[:NUM_MASK].astype(jnp.int32)
    flags = jnp.isin(jnp.arange(M, dtype=jnp.int32),
                     masked_clusters).astype(jnp.float32).reshape(1, M)

    pc = _run_fps(pt3, p0)
    pcT = pc.T

    bf = jnp.bfloat16
    xm, mk = _run_cluster(p0, x0, pcT, flags)
    fc1 = _run_knn(pc, pt2, pt2.astype(bf), xm.astype(bf), W_enc.astype(bf),
                   b_enc.reshape(1, -1), w1.astype(bf))
    z1, s1, q1 = _run_interp(p0, pcT, fc1.astype(bf), b1.reshape(1, -1))
    z2, s2, q2 = _run_bn_mm(z1, s1, q1, g1.reshape(1, -1), beta1.reshape(1, -1),
                            w2.astype(bf), b2.reshape(1, -1), 256, 128)
    xrec, lsum, msum = _run_final(z2, s2, q2, g2.reshape(1, -1),
                                  beta2.reshape(1, -1), w3.astype(bf),
                                  b3.reshape(1, -1), xm, mk)

    loss = lsum[0, 0] / (msum[0, 0] * 6.0)
    mask = mk[:, 0] > 0.5
    visible = jnp.logical_not(mask)
    return (loss, xrec, mask, visible, xm)


# speedup vs baseline: 1.6659x; 1.0025x over previous
import jax
import jax.numpy as jnp
from jax import lax
from jax.experimental import pallas as pl
from jax.experimental.pallas import tpu as pltpu

N = 24000
M = 64
COARSE = 512
NUM_MASK = 19
KNN = 16
SUB = 8
LANES = N // SUB
BLK = 480
NBLK = N // BLK
F_INF = 3e38
IBIG = 1 << 30

def _dot_b(a, b):
    return jax.lax.dot_general(
        a, b, (((1,), (0,)), ((), ())), preferred_element_type=jnp.float32)


def _dot_bf(a, b):
    return jax.lax.dot_general(
        a.astype(jnp.bfloat16), b.astype(jnp.bfloat16), (((1,), (0,)), ((), ())),
        preferred_element_type=jnp.float32)


def _fps_body(pt_ref, p0_ref, pc_ref, dist_ref):
    dist_ref[...] = jnp.full((SUB, LANES), 1e10, jnp.float32)
    ii = (lax.broadcasted_iota(jnp.int32, (SUB, LANES), 0) * LANES
          + lax.broadcasted_iota(jnp.int32, (SUB, LANES), 1))

    def body(k, last):
        row = p0_ref[pl.ds(last, 1), :]
        pc_ref[pl.ds(k, 1), :] = row
        cx, cy, cz = row[:, 0:1], row[:, 1:2], row[:, 2:3]
        dx = pt_ref[0] - cx
        dy = pt_ref[1] - cy
        dz = pt_ref[2] - cz
        d = (dx * dx + dy * dy) + dz * dz
        dist = jnp.minimum(dist_ref[...], d)
        dist_ref[...] = dist
        m = jnp.max(dist, axis=(0, 1), keepdims=True)
        nxt = jnp.min(jnp.where(dist == m, ii, IBIG))
        return nxt

    last = lax.fori_loop(0, COARSE - 1, body, jnp.int32(0), unroll=2)
    pc_ref[pl.ds(COARSE - 1, 1), :] = p0_ref[pl.ds(last, 1), :]


def _run_fps(pt3, p0):
    return pl.pallas_call(
        _fps_body,
        out_shape=jax.ShapeDtypeStruct((COARSE, 3), jnp.float32),
        scratch_shapes=[pltpu.VMEM((SUB, LANES), jnp.float32)],
    )(pt3, p0)


def _cluster_body(p_ref, x_ref, pcT_ref, fl_ref, xm_ref, mk_ref):
    p = p_ref[...]
    px, py, pz = p[:, 0:1], p[:, 1:2], p[:, 2:3]
    sp = (px * px + py * py) + pz * pz
    cx = pcT_ref[0:1, 0:M]
    cy = pcT_ref[1:2, 0:M]
    cz = pcT_ref[2:3, 0:M]
    sc = (cx * cx + cy * cy) + cz * cz
    dot = _dot_bf(p, pcT_ref[:, 0:M])
    d2 = (sp + sc) - 2.0 * dot
    rmin = jnp.min(d2, axis=1, keepdims=True)
    i64 = lax.broadcasted_iota(jnp.int32, d2.shape, 1)
    cl = jnp.min(jnp.where(d2 == rmin, i64, IBIG), axis=1, keepdims=True)
    onehot = i64 == cl
    maskf = jnp.max(jnp.where(onehot, fl_ref[0:1, :], 0.0), axis=1,
                    keepdims=True)
    mk_ref[...] = maskf
    xm_ref[...] = jnp.where(maskf > 0.0, 0.0, x_ref[...])


def _run_cluster(p0, x0, pcT, flags):
    blk = 3000
    grid = N // blk
    return pl.pallas_call(
        _cluster_body,
        grid=(grid,),
        in_specs=[
            pl.BlockSpec((blk, 3), lambda i: (i, 0)),
            pl.BlockSpec((blk, 6), lambda i: (i, 0)),
            pl.BlockSpec((3, COARSE), lambda i: (0, 0)),
            pl.BlockSpec((1, M), lambda i: (0, 0)),
        ],
        out_specs=[
            pl.BlockSpec((blk, 6), lambda i: (i, 0)),
            pl.BlockSpec((blk, 1), lambda i: (i, 0)),
        ],
        out_shape=[
            jax.ShapeDtypeStruct((N, 6), jnp.float32),
            jax.ShapeDtypeStruct((N, 1), jnp.float32),
        ],
    )(p0, x0, pcT, flags)


G = 64
GL = N // G
RB = 16
TOPG = 4


def _knn_body(pc_ref, pt_ref, ptb_ref, xm_ref, we_ref, be_ref, w1_ref, fc1_ref,
              d2_ref, x_ref, t_ref, it_ref, sp_ref):
    @pl.when(pl.program_id(0) == 0)
    def _():
        px = pt_ref[0:1, :]
        py = pt_ref[1:2, :]
        pz = pt_ref[2:3, :]
        sp_ref[...] = (px * px + py * py) + pz * pz

    c = pc_ref[...]
    cx, cy, cz = c[:, 0:1], c[:, 1:2], c[:, 2:3]
    sc = (cx * cx + cy * cy) + cz * cz
    dot = jax.lax.dot_general(
        c.astype(jnp.bfloat16), ptb_ref[...], (((1,), (0,)), ((), ())),
        preferred_element_type=jnp.float32)
    d2 = (sc + sp_ref[...]) - 2.0 * dot
    d2_ref[...] = d2

    x_ref[...] = d2.reshape(RB, G, GL)
    cands = []
    for j in range(TOPG):
        x = x_ref[...]
        m = jnp.min(x, axis=2, keepdims=True)
        if j + 1 < TOPG:
            x_ref[...] = jnp.where(x == m, F_INF, x)
        cands.append(m)
    cand = jnp.concatenate(cands, axis=2).reshape(RB, G * TOPG)

    t = cand
    for j in range(KNN):
        m = jnp.min(cand, axis=1, keepdims=True)
        if j + 1 < KNN:
            cand = jnp.where(cand == m, F_INF, cand)
        t = m
    t_ref[...] = t
    it_ref[...] = jnp.full((RB, 1), IBIG - 1, jnp.int32)

    cnt = jnp.sum((d2 <= t).astype(jnp.float32), axis=1, keepdims=True)
    bad = jnp.max(jnp.abs(cnt - KNN)) > 0.0

    @pl.when(bad)
    def _():
        x_ref[...] = d2_ref[...].reshape(RB, G, GL)
        ig = (lax.broadcasted_iota(jnp.int32, (RB, G, GL), 1) * GL
              + lax.broadcasted_iota(jnp.int32, (RB, G, GL), 2))

        def fb_body(_, carry):
            x = x_ref[...]
            rmin = jnp.min(x, axis=(1, 2), keepdims=True)
            imin = jnp.min(jnp.where(x == rmin, ig, IBIG), axis=(1, 2),
                           keepdims=True)
            x_ref[...] = jnp.where(ig == imin, F_INF, x)
            return rmin[:, 0, :], imin[:, 0, :]
        t_fb, it_fb = lax.fori_loop(
            0, KNN, fb_body,
            (jnp.zeros((RB, 1), jnp.float32), jnp.zeros((RB, 1), jnp.int32)))
        t_ref[...] = t_fb
        it_ref[...] = it_fb

    iN = lax.broadcasted_iota(jnp.int32, (RB, N), 1)
    d2o = d2_ref[...]
    sel = (d2o < t_ref[...]) | ((d2o == t_ref[...]) & (iN <= it_ref[...]))
    A = jnp.where(sel, 1.0 / KNN, 0.0).astype(jnp.bfloat16)
    agg = _dot_b(A, xm_ref[...])
    f = jnp.maximum(_dot_b(agg.astype(jnp.bfloat16), we_ref[...])
                    + be_ref[0:1, :], 0.0)
    fc1_ref[...] = _dot_b(f.astype(jnp.bfloat16), w1_ref[...])


def _run_knn(pc, pt2, ptb, xm, W_enc, b_enc, w1):
    return pl.pallas_call(
        _knn_body,
        grid=(COARSE // RB,),
        in_specs=[
            pl.BlockSpec((RB, 3), lambda i: (i, 0)),
            pl.BlockSpec((3, N), lambda i: (0, 0)),
            pl.BlockSpec((3, N), lambda i: (0, 0)),
            pl.BlockSpec((N, 6), lambda i: (0, 0)),
            pl.BlockSpec((6, 512), lambda i: (0, 0)),
            pl.BlockSpec((1, 512), lambda i: (0, 0)),
            pl.BlockSpec((512, 256), lambda i: (0, 0)),
        ],
        out_specs=pl.BlockSpec((RB, 256), lambda i: (i, 0)),
        out_shape=jax.ShapeDtypeStruct((COARSE, 256), jnp.float32),
        scratch_shapes=[pltpu.VMEM((RB, N), jnp.float32),
                        pltpu.VMEM((RB, G, GL), jnp.float32),
                        pltpu.VMEM((RB, 1), jnp.float32),
                        pltpu.VMEM((RB, 1), jnp.int32),
                        pltpu.VMEM((1, N), jnp.float32)],
    )(pc, pt2, ptb, xm, W_enc, b_enc, w1)


def _interp_body(p_ref, pcT_ref, fc1_ref, b1_ref, z1_ref, s1_ref, q1_ref):
    @pl.when(pl.program_id(0) == 0)
    def _():
        s1_ref[...] = jnp.zeros_like(s1_ref)
        q1_ref[...] = jnp.zeros_like(q1_ref)

    p = p_ref[...]
    px, py, pz = p[:, 0:1], p[:, 1:2], p[:, 2:3]
    sp = (px * px + py * py) + pz * pz
    cx = pcT_ref[0:1, :]
    cy = pcT_ref[1:2, :]
    cz = pcT_ref[2:3, :]
    sc = (cx * cx + cy * cy) + cz * cz
    dot = _dot_bf(p, pcT_ref[...])
    d2 = (sp + sc) - 2.0 * dot

    ii = lax.broadcasted_iota(jnp.int32, d2.shape, 1)
    ds_ = []
    is_ = []
    for j in range(3):
        rmin = jnp.min(d2, axis=1, keepdims=True)
        imin = jnp.min(jnp.where(d2 == rmin, ii, IBIG), axis=1, keepdims=True)
        ds_.append(rmin)
        is_.append(imin)
        if j < 2:
            d2 = jnp.where(ii == imin, F_INF, d2)

    w = [1.0 / (jnp.sqrt(jnp.maximum(d, 1e-12)) + 1e-8) for d in ds_]
    wsum = (w[0] + w[1]) + w[2]
    wmat = (jnp.where(ii == is_[0], w[0] / wsum, 0.0)
            + jnp.where(ii == is_[1], w[1] / wsum, 0.0)
            + jnp.where(ii == is_[2], w[2] / wsum, 0.0))
    z1 = _dot_b(wmat.astype(jnp.bfloat16), fc1_ref[...]) + b1_ref[0:1, :]
    z1_ref[...] = z1
    s1_ref[...] += jnp.sum(z1, axis=0, keepdims=True)
    q1_ref[...] += jnp.sum(z1 * z1, axis=0, keepdims=True)


def _run_interp(p0, pcT, fc1, b1):
    return pl.pallas_call(
        _interp_body,
        grid=(NBLK,),
        in_specs=[
            pl.BlockSpec((BLK, 3), lambda i: (i, 0)),
            pl.BlockSpec((3, COARSE), lambda i: (0, 0)),
            pl.BlockSpec((COARSE, 256), lambda i: (0, 0)),
            pl.BlockSpec((1, 256), lambda i: (0, 0)),
        ],
        out_specs=[
            pl.BlockSpec((BLK, 256), lambda i: (i, 0)),
            pl.BlockSpec((1, 256), lambda i: (0, 0)),
            pl.BlockSpec((1, 256), lambda i: (0, 0)),
        ],
        out_shape=[
            jax.ShapeDtypeStruct((N, 256), jnp.float32),
            jax.ShapeDtypeStruct((1, 256), jnp.float32),
            jax.ShapeDtypeStruct((1, 256), jnp.float32),
        ],
    )(p0, pcT, fc1, b1)


def _bn_mm_body(z_ref, s_ref, q_ref, g_ref, be_ref, w_ref, bb_ref,
                o_ref, so_ref, qo_ref):
    @pl.when(pl.program_id(0) == 0)
    def _():
        so_ref[...] = jnp.zeros_like(so_ref)
        qo_ref[...] = jnp.zeros_like(qo_ref)

    inv_n = jnp.float32(1.0 / N)
    mu = s_ref[...] * inv_n
    var = q_ref[...] * inv_n - mu * mu
    z = z_ref[...]
    h = (z - mu) / jnp.sqrt(var + 1e-5) * g_ref[0:1, :] + be_ref[0:1, :]
    h = jnp.maximum(h, 0.0)
    o = _dot_b(h.astype(jnp.bfloat16), w_ref[...]) + bb_ref[0:1, :]
    o_ref[...] = o
    so_ref[...] += jnp.sum(o, axis=0, keepdims=True)
    qo_ref[...] += jnp.sum(o * o, axis=0, keepdims=True)


def _run_bn_mm(z, s, q, g, beta, w, b, din, dout):
    return pl.pallas_call(
        _bn_mm_body,
        grid=(NBLK,),
        in_specs=[
            pl.BlockSpec((BLK, din), lambda i: (i, 0)),
            pl.BlockSpec((1, din), lambda i: (0, 0)),
            pl.BlockSpec((1, din), lambda i: (0, 0)),
            pl.BlockSpec((1, din), lambda i: (0, 0)),
            pl.BlockSpec((1, din), lambda i: (0, 0)),
            pl.BlockSpec((din, dout), lambda i: (0, 0)),
            pl.BlockSpec((1, dout), lambda i: (0, 0)),
        ],
        out_specs=[
            pl.BlockSpec((BLK, dout), lambda i: (i, 0)),
            pl.BlockSpec((1, dout), lambda i: (0, 0)),
            pl.BlockSpec((1, dout), lambda i: (0, 0)),
        ],
        out_shape=[
            jax.ShapeDtypeStruct((N, dout), jnp.float32),
            jax.ShapeDtypeStruct((1, dout), jnp.float32),
            jax.ShapeDtypeStruct((1, dout), jnp.float32),
        ],
    )(z, s, q, g, beta, w, b)


def _final_body(z_ref, s_ref, q_ref, g_ref, be_ref, w_ref, bb_ref,
                xm_ref, mk_ref, xr_ref, ls_ref, ms_ref):
    @pl.when(pl.program_id(0) == 0)
    def _():
        ls_ref[...] = jnp.zeros_like(ls_ref)
        ms_ref[...] = jnp.zeros_like(ms_ref)

    inv_n = jnp.float32(1.0 / N)
    mu = s_ref[...] * inv_n
    var = q_ref[...] * inv_n - mu * mu
    h = (z_ref[...] - mu) / jnp.sqrt(var + 1e-5) * g_ref[0:1, :] + be_ref[0:1, :]
    h = jnp.maximum(h, 0.0)
    xr = _dot_b(h.astype(jnp.bfloat16), w_ref[...]) + bb_ref[0:1, :]
    xm = xm_ref[...]
    mk = mk_ref[...]
    l1 = jnp.abs(xr - xm) * mk
    ls_ref[...] += jnp.sum(l1, axis=(0, 1), keepdims=True)
    ms_ref[...] += jnp.sum(mk, axis=(0, 1), keepdims=True)
    xr_ref[...] = jnp.where(mk > 0.0, xr, xm)


def _run_final(z2, s2, q2, g2, beta2, w3, b3, xm, mk):
    return pl.pallas_call(
        _final_body,
        grid=(NBLK,),
        in_specs=[
            pl.BlockSpec((BLK, 128), lambda i: (i, 0)),
            pl.BlockSpec((1, 128), lambda i: (0, 0)),
            pl.BlockSpec((1, 128), lambda i: (0, 0)),
            pl.BlockSpec((1, 128), lambda i: (0, 0)),
            pl.BlockSpec((1, 128), lambda i: (0, 0)),
            pl.BlockSpec((128, 6), lambda i: (0, 0)),
            pl.BlockSpec((1, 6), lambda i: (0, 0)),
            pl.BlockSpec((BLK, 6), lambda i: (i, 0)),
            pl.BlockSpec((BLK, 1), lambda i: (i, 0)),
        ],
        out_specs=[
            pl.BlockSpec((BLK, 6), lambda i: (i, 0)),
            pl.BlockSpec((1, 1), lambda i: (0, 0)),
            pl.BlockSpec((1, 1), lambda i: (0, 0)),
        ],
        out_shape=[
            jax.ShapeDtypeStruct((N, 6), jnp.float32),
            jax.ShapeDtypeStruct((1, 1), jnp.float32),
            jax.ShapeDtypeStruct((1, 1), jnp.float32),
        ],
    )(z2, s2, q2, g2, beta2, w3, b3, xm, mk)


def kernel(point, features, W_enc, b_enc, w1, b1, g1, beta1, w2, b2, g2,
           beta2, w3, b3):
    p0 = point.reshape(-1, 3)
    x0 = features.reshape(-1, 6)
    pt2 = p0.T
    pt3 = pt2.reshape(3, SUB, LANES)

    masked_clusters = jax.random.permutation(
        jax.random.key(1), M)[:NUM_MASK].astype(jnp.int32)
    flags = jnp.isin(jnp.arange(M, dtype=jnp.int32),
                     masked_clusters).astype(jnp.float32).reshape(1, M)

    pc = _run_fps(pt3, p0)
    pcT = pc.T

    bf = jnp.bfloat16
    xm, mk = _run_cluster(p0, x0, pcT, flags)
    fc1 = _run_knn(pc, pt2, pt2.astype(bf), xm.astype(bf), W_enc.astype(bf),
                   b_enc.reshape(1, -1), w1.astype(bf))
    z1, s1, q1 = _run_interp(p0, pcT, fc1.astype(bf), b1.reshape(1, -1))
    z2, s2, q2 = _run_bn_mm(z1, s1, q1, g1.reshape(1, -1), beta1.reshape(1, -1),
                            w2.astype(bf), b2.reshape(1, -1), 256, 128)
    xrec, lsum, msum = _run_final(z2, s2, q2, g2.reshape(1, -1),
                                  beta2.reshape(1, -1), w3.astype(bf),
                                  b3.reshape(1, -1), xm, mk)

    loss = lsum[0, 0] / (msum[0, 0] * 6.0)
    mask = mk[:, 0] > 0.5
    visible = jnp.logical_not(mask)
    return (loss, xrec, mask, visible, xm)
